# Initial kernel scaffold; baseline (speedup 1.0000x reference)
import jax
import jax.numpy as jnp
from jax.experimental import pallas as pl


def kernel(vertices, faces, face_probs):
    raise NotImplementedError("write your pallas kernel here")


# speedup vs baseline: 8.3377x; 8.3377x over previous
import functools
import jax
import jax.numpy as jnp
from jax import lax
from jax.experimental import pallas as pl
from jax.experimental.pallas import tpu as pltpu
from jax.experimental.pallas import tpu_sc as plsc

_NS = 10
_FPAD = 5120
_NC = 2
_NSUB = 16
_NW = _NC * _NSUB
_FPW = _FPAD // _NW
_CHUNKS = _FPW // 16

_I_BLK = 256
_J_BLK = 512
_NI = _FPAD // _I_BLK
_NJ = _FPAD // _J_BLK


def _sc_body(vx_h, vy_h, vz_h, f0_h, f1_h, f2_h, al_h, be_h, ga_h,
             pt_h, w_h, cit_h, cj_h,
             vx, vy, vz, f0b, f1b, f2b, alb, ptb, wb, citb, cjb):
    wid = lax.axis_index("s") * _NC + lax.axis_index("c")
    tb = wid * _FPW

    pltpu.sync_copy(vx_h, vx)
    pltpu.sync_copy(vy_h, vy)
    pltpu.sync_copy(vz_h, vz)
    pltpu.sync_copy(f0_h.at[pl.ds(tb, _FPW)], f0b)
    pltpu.sync_copy(f1_h.at[pl.ds(tb, _FPW)], f1b)
    pltpu.sync_copy(f2_h.at[pl.ds(tb, _FPW)], f2b)
    for s in range(_NS):
        pltpu.sync_copy(al_h.at[pl.ds(s * _FPAD + tb, _FPW)],
                        alb.at[pl.ds(s * _FPW, _FPW)])
        pltpu.sync_copy(be_h.at[pl.ds(s * _FPAD + tb, _FPW)],
                        alb.at[pl.ds((_NS + s) * _FPW, _FPW)])
        pltpu.sync_copy(ga_h.at[pl.ds(s * _FPAD + tb, _FPW)],
                        alb.at[pl.ds((2 * _NS + s) * _FPW, _FPW)])

    one = jnp.full((16,), 1.0, jnp.float32)
    third = jnp.float32(1.0 / 3.0)

    def chunk(k, _):
        o = k * 16
        f0v = f0b[pl.ds(o, 16)]
        f1v = f1b[pl.ds(o, 16)]
        f2v = f2b[pl.ds(o, 16)]
        v1x = plsc.load_gather(vx, [f0v])
        v1y = plsc.load_gather(vy, [f0v])
        v1z = plsc.load_gather(vz, [f0v])
        v2x = plsc.load_gather(vx, [f1v])
        v2y = plsc.load_gather(vy, [f1v])
        v2z = plsc.load_gather(vz, [f1v])
        v3x = plsc.load_gather(vx, [f2v])
        v3y = plsc.load_gather(vy, [f2v])
        v3z = plsc.load_gather(vz, [f2v])

        e0x = v2x - v1x
        e0y = v2y - v1y
        e0z = v2z - v1z
        e1x = v3x - v1x
        e1y = v3y - v1y
        e1z = v3z - v1z
        d00 = e0x * e0x + e0y * e0y + e0z * e0z
        d01 = e0x * e1x + e0y * e1y + e0z * e1z
        d11 = e1x * e1x + e1y * e1y + e1z * e1z
        den = d00 * d11 - d01 * d01
        c0 = v1x * e0x + v1y * e0y + v1z * e0z
        c1 = v1x * e1x + v1y * e1y + v1z * e1z
        sgn = jnp.where(den > 0.0, 1.0, -1.0).astype(jnp.float32)
        dnz = den != 0.0
        sd00 = sgn * d00
        sd01 = sgn * d01
        sd11 = sgn * d11
        ax = sd11 * e0x - sd01 * e1x
        ay = sd11 * e0y - sd01 * e1y
        az = sd11 * e0z - sd01 * e1z
        a4 = sd01 * c1 - sd11 * c0
        bx = sd00 * e1x - sd01 * e0x
        by = sd00 * e1y - sd01 * e0y
        bz = sd00 * e1z - sd01 * e0z
        b4 = sd01 * c0 - sd00 * c1
        zero = jnp.zeros((16,), jnp.float32)
        cx = jnp.where(dnz, ax + bx, zero)
        cy = jnp.where(dnz, ay + by, zero)
        cz = jnp.where(dnz, az + bz, zero)
        c4 = jnp.where(dnz, a4 + b4 - sgn * den, one)

        cenx = (v1x + v2x + v3x) * third
        ceny = (v1y + v2y + v3y) * third
        cenz = (v1z + v2z + v3z) * third
        sq = cenx * cenx + ceny * ceny + cenz * cenz
        gid = tb + o + lax.iota(jnp.int32, 16)
        sqm = jnp.where(gid < jnp.int32(5000), sq, jnp.float32(1e9))

        wb[pl.ds(0 * _FPW + o, 16)] = ax
        wb[pl.ds(1 * _FPW + o, 16)] = ay
        wb[pl.ds(2 * _FPW + o, 16)] = az
        wb[pl.ds(3 * _FPW + o, 16)] = a4
        wb[pl.ds(4 * _FPW + o, 16)] = bx
        wb[pl.ds(5 * _FPW + o, 16)] = by
        wb[pl.ds(6 * _FPW + o, 16)] = bz
        wb[pl.ds(7 * _FPW + o, 16)] = b4
        wb[pl.ds(8 * _FPW + o, 16)] = cx
        wb[pl.ds(9 * _FPW + o, 16)] = cy
        wb[pl.ds(10 * _FPW + o, 16)] = cz
        wb[pl.ds(11 * _FPW + o, 16)] = c4

        citb[pl.ds(0 * _FPW + o, 16)] = cenx
        citb[pl.ds(1 * _FPW + o, 16)] = ceny
        citb[pl.ds(2 * _FPW + o, 16)] = cenz
        citb[pl.ds(3 * _FPW + o, 16)] = sqm
        citb[pl.ds(4 * _FPW + o, 16)] = one

        m2 = jnp.float32(-2.0)
        cjb[pl.ds(0 * _FPW + o, 16)] = m2 * cenx
        cjb[pl.ds(1 * _FPW + o, 16)] = m2 * ceny
        cjb[pl.ds(2 * _FPW + o, 16)] = m2 * cenz
        cjb[pl.ds(3 * _FPW + o, 16)] = one
        cjb[pl.ds(4 * _FPW + o, 16)] = sqm

        for s in range(_NS):
            al = alb[pl.ds(s * _FPW + o, 16)]
            be = alb[pl.ds((_NS + s) * _FPW + o, 16)]
            ga = alb[pl.ds((2 * _NS + s) * _FPW + o, 16)]
            px = al * v1x + be * v2x + ga * v3x
            py = al * v1y + be * v2y + ga * v3y
            pz = al * v1z + be * v2z + ga * v3z
            ptb[pl.ds((s * 4 + 0) * _FPW + o, 16)] = px
            ptb[pl.ds((s * 4 + 1) * _FPW + o, 16)] = py
            ptb[pl.ds((s * 4 + 2) * _FPW + o, 16)] = pz
            ptb[pl.ds((s * 4 + 3) * _FPW + o, 16)] = one
        return ()

    lax.fori_loop(0, _CHUNKS, chunk, ())

    for r in range(_NS * 4):
        pltpu.sync_copy(ptb.at[pl.ds(r * _FPW, _FPW)],
                        pt_h.at[pl.ds(r * _FPAD + tb, _FPW)])
    for r in range(12):
        pltpu.sync_copy(wb.at[pl.ds(r * _FPW, _FPW)],
                        w_h.at[pl.ds(r * _FPAD + tb, _FPW)])
    for r in range(5):
        pltpu.sync_copy(citb.at[pl.ds(r * _FPW, _FPW)],
                        cit_h.at[pl.ds(r * _FPAD + tb, _FPW)])
        pltpu.sync_copy(cjb.at[pl.ds(r * _FPW, _FPW)],
                        cj_h.at[pl.ds(r * _FPAD + tb, _FPW)])


_sc_stage1 = functools.partial(
    pl.kernel,
    _sc_body,
    out_type=[
        jax.ShapeDtypeStruct((_NS * 4 * _FPAD,), jnp.float32),
        jax.ShapeDtypeStruct((12 * _FPAD,), jnp.float32),
        jax.ShapeDtypeStruct((5 * _FPAD,), jnp.float32),
        jax.ShapeDtypeStruct((5 * _FPAD,), jnp.float32),
    ],
    mesh=plsc.VectorSubcoreMesh(
        core_axis_name="c", subcore_axis_name="s",
        num_cores=_NC, num_subcores=_NSUB),
    compiler_params=pltpu.CompilerParams(needs_layout_passes=False),
    scratch_types=[
        pltpu.VMEM((2500,), jnp.float32),
        pltpu.VMEM((2500,), jnp.float32),
        pltpu.VMEM((2500,), jnp.float32),
        pltpu.VMEM((_FPW,), jnp.int32),
        pltpu.VMEM((_FPW,), jnp.int32),
        pltpu.VMEM((_FPW,), jnp.int32),
        pltpu.VMEM((3 * _NS * _FPW,), jnp.float32),
        pltpu.VMEM((_NS * 4 * _FPW,), jnp.float32),
        pltpu.VMEM((12 * _FPW,), jnp.float32),
        pltpu.VMEM((5 * _FPW,), jnp.float32),
        pltpu.VMEM((5 * _FPW,), jnp.float32),
    ],
)


def _tc_body(pt_ref, w_ref, cit_ref, cj_ref, fp_ref, out_ref):
    i = pl.program_id(0)
    j = pl.program_id(1)
    dn = (((0,), (0,)), ((), ()))
    hi = lax.Precision.HIGHEST
    d2 = lax.dot_general(cit_ref[...], cj_ref[...], dn, precision=hi)
    ri = lax.broadcasted_iota(jnp.int32, (_I_BLK, _J_BLK), 0) + i * _I_BLK
    rj = lax.broadcasted_iota(jnp.int32, (_I_BLK, _J_BLK), 1) + j * _J_BLK
    mask = (d2 < 1.0) & (ri != rj)
    wcat = jnp.concatenate([w_ref[0], w_ref[1], w_ref[2]], axis=1)
    has = jnp.zeros((_I_BLK, _J_BLK), jnp.bool_)
    for s in range(_NS):
        r = lax.dot_general(pt_ref[s], wcat, dn, precision=hi)
        nv = r[:, :_J_BLK]
        nw = r[:, _J_BLK:2 * _J_BLK]
        t = r[:, 2 * _J_BLK:]
        has |= (nv >= 0.0) & (nw >= 0.0) & (t <= 0.0)
    partial = jnp.sum(jnp.where(mask & has, fp_ref[...], 0.0))

    @pl.when((i == 0) & (j == 0))
    def _():
        out_ref[0, 0] = 0.0
    acc = out_ref[0, 0] + partial

    @pl.when((i == _NI - 1) & (j == _NJ - 1))
    def _():
        out_ref[0, 0] = acc * (1.0 / 5000.0)

    @pl.when(~((i == _NI - 1) & (j == _NJ - 1)))
    def _():
        out_ref[0, 0] = acc


def _stage2(PT, W, CIT, CJ, fp2):
    out = pl.pallas_call(
        _tc_body,
        grid=(_NI, _NJ),
        in_specs=[
            pl.BlockSpec((_NS, 4, _I_BLK), lambda i, j: (0, 0, i)),
            pl.BlockSpec((3, 4, _J_BLK), lambda i, j: (0, 0, j)),
            pl.BlockSpec((5, _I_BLK), lambda i, j: (0, i)),
            pl.BlockSpec((5, _J_BLK), lambda i, j: (0, j)),
            pl.BlockSpec((_I_BLK, 1), lambda i, j: (i, 0)),
        ],
        out_specs=pl.BlockSpec(memory_space=pltpu.SMEM),
        out_shape=jax.ShapeDtypeStruct((1, 1), jnp.float32),
    )(PT, W, CIT, CJ, fp2)
    return out[0, 0]


@jax.jit
def kernel(vertices, faces, face_probs):
    F = faces.shape[0]
    kk = jax.random.key(42)
    ka, kb = jax.random.split(kk)
    alpha = jax.random.uniform(ka, (F, _NS), dtype=jnp.float32)
    beta = jax.random.uniform(kb, (F, _NS), dtype=jnp.float32) * (1.0 - alpha)
    gamma = 1.0 - alpha - beta
    pad = _FPAD - F
    alf = jnp.pad(alpha, ((0, pad), (0, 0))).T.reshape(-1)
    bef = jnp.pad(beta, ((0, pad), (0, 0))).T.reshape(-1)
    gaf = jnp.pad(gamma, ((0, pad), (0, 0))).T.reshape(-1)
    facesP = jnp.pad(faces, ((0, pad), (0, 0)))
    vx = vertices[:, 0]
    vy = vertices[:, 1]
    vz = vertices[:, 2]
    ptf, wf, citf, cjf = _sc_stage1()(
        vx, vy, vz,
        facesP[:, 0], facesP[:, 1], facesP[:, 2],
        alf, bef, gaf)
    PT = ptf.reshape(_NS, 4, _FPAD)
    W = wf.reshape(3, 4, _FPAD)
    CIT = citf.reshape(5, _FPAD)
    CJ = cjf.reshape(5, _FPAD)
    fp2 = jnp.pad(face_probs, (0, pad)).reshape(_FPAD, 1)
    loss = _stage2(PT, W, CIT, CJ, fp2)
    return loss


# speedup vs baseline: 11.8729x; 1.4240x over previous
import functools
import jax
import jax.numpy as jnp
from jax import lax
from jax.experimental import pallas as pl
from jax.experimental.pallas import tpu as pltpu
from jax.experimental.pallas import tpu_sc as plsc

_NS = 10
_FPAD = 5120
_NC = 2
_NSUB = 16
_NW = _NC * _NSUB
_FPW = _FPAD // _NW
_CHUNKS = _FPW // 16

_I_BLK = 256
_J_BLK = 512
_NI = _FPAD // _I_BLK
_NJ = _FPAD // _J_BLK


def _sc_body(vx_h, vy_h, vz_h, f0_h, f1_h, f2_h, al_h, be_h, ga_h,
             pt_h, w_h, cit_h, cj_h,
             vx, vy, vz, f0b, f1b, f2b, alb, ptb, wb, citb, cjb):
    wid = lax.axis_index("s") * _NC + lax.axis_index("c")
    tb = wid * _FPW

    pltpu.sync_copy(vx_h, vx)
    pltpu.sync_copy(vy_h, vy)
    pltpu.sync_copy(vz_h, vz)
    pltpu.sync_copy(f0_h.at[pl.ds(tb, _FPW)], f0b)
    pltpu.sync_copy(f1_h.at[pl.ds(tb, _FPW)], f1b)
    pltpu.sync_copy(f2_h.at[pl.ds(tb, _FPW)], f2b)
    for s in range(_NS):
        pltpu.sync_copy(al_h.at[pl.ds(s * _FPAD + tb, _FPW)],
                        alb.at[pl.ds(s * _FPW, _FPW)])
        pltpu.sync_copy(be_h.at[pl.ds(s * _FPAD + tb, _FPW)],
                        alb.at[pl.ds((_NS + s) * _FPW, _FPW)])
        pltpu.sync_copy(ga_h.at[pl.ds(s * _FPAD + tb, _FPW)],
                        alb.at[pl.ds((2 * _NS + s) * _FPW, _FPW)])

    one = jnp.full((16,), 1.0, jnp.float32)
    third = jnp.float32(1.0 / 3.0)

    def chunk(k, _):
        o = k * 16
        f0v = f0b[pl.ds(o, 16)]
        f1v = f1b[pl.ds(o, 16)]
        f2v = f2b[pl.ds(o, 16)]
        v1x = plsc.load_gather(vx, [f0v])
        v1y = plsc.load_gather(vy, [f0v])
        v1z = plsc.load_gather(vz, [f0v])
        v2x = plsc.load_gather(vx, [f1v])
        v2y = plsc.load_gather(vy, [f1v])
        v2z = plsc.load_gather(vz, [f1v])
        v3x = plsc.load_gather(vx, [f2v])
        v3y = plsc.load_gather(vy, [f2v])
        v3z = plsc.load_gather(vz, [f2v])

        e0x = v2x - v1x
        e0y = v2y - v1y
        e0z = v2z - v1z
        e1x = v3x - v1x
        e1y = v3y - v1y
        e1z = v3z - v1z
        d00 = e0x * e0x + e0y * e0y + e0z * e0z
        d01 = e0x * e1x + e0y * e1y + e0z * e1z
        d11 = e1x * e1x + e1y * e1y + e1z * e1z
        den = d00 * d11 - d01 * d01
        c0 = v1x * e0x + v1y * e0y + v1z * e0z
        c1 = v1x * e1x + v1y * e1y + v1z * e1z
        sgn = jnp.where(den > 0.0, 1.0, -1.0).astype(jnp.float32)
        dnz = den != 0.0
        sd00 = sgn * d00
        sd01 = sgn * d01
        sd11 = sgn * d11
        ax = sd11 * e0x - sd01 * e1x
        ay = sd11 * e0y - sd01 * e1y
        az = sd11 * e0z - sd01 * e1z
        a4 = sd01 * c1 - sd11 * c0
        bx = sd00 * e1x - sd01 * e0x
        by = sd00 * e1y - sd01 * e0y
        bz = sd00 * e1z - sd01 * e0z
        b4 = sd01 * c0 - sd00 * c1
        dds = jnp.where(dnz, sgn * den, jnp.float32(-1.0))

        cenx = (v1x + v2x + v3x) * third
        ceny = (v1y + v2y + v3y) * third
        cenz = (v1z + v2z + v3z) * third
        sq = cenx * cenx + ceny * ceny + cenz * cenz
        gid = tb + o + lax.iota(jnp.int32, 16)
        sqm = jnp.where(gid < jnp.int32(5000), sq, jnp.float32(1e9))

        wb[pl.ds(0 * _FPW + o, 16)] = ax
        wb[pl.ds(1 * _FPW + o, 16)] = ay
        wb[pl.ds(2 * _FPW + o, 16)] = az
        wb[pl.ds(3 * _FPW + o, 16)] = a4
        wb[pl.ds(4 * _FPW + o, 16)] = bx
        wb[pl.ds(5 * _FPW + o, 16)] = by
        wb[pl.ds(6 * _FPW + o, 16)] = bz
        wb[pl.ds(7 * _FPW + o, 16)] = b4

        citb[pl.ds(0 * _FPW + o, 16)] = cenx
        citb[pl.ds(1 * _FPW + o, 16)] = ceny
        citb[pl.ds(2 * _FPW + o, 16)] = cenz
        citb[pl.ds(3 * _FPW + o, 16)] = sqm
        citb[pl.ds(4 * _FPW + o, 16)] = one
        citb[pl.ds(5 * _FPW + o, 16)] = jnp.zeros((16,), jnp.float32)

        m2 = jnp.float32(-2.0)
        cjb[pl.ds(0 * _FPW + o, 16)] = m2 * cenx
        cjb[pl.ds(1 * _FPW + o, 16)] = m2 * ceny
        cjb[pl.ds(2 * _FPW + o, 16)] = m2 * cenz
        cjb[pl.ds(3 * _FPW + o, 16)] = one
        cjb[pl.ds(4 * _FPW + o, 16)] = sqm
        cjb[pl.ds(5 * _FPW + o, 16)] = dds

        for s in range(_NS):
            al = alb[pl.ds(s * _FPW + o, 16)]
            be = alb[pl.ds((_NS + s) * _FPW + o, 16)]
            ga = alb[pl.ds((2 * _NS + s) * _FPW + o, 16)]
            px = al * v1x + be * v2x + ga * v3x
            py = al * v1y + be * v2y + ga * v3y
            pz = al * v1z + be * v2z + ga * v3z
            ptb[pl.ds((s * 4 + 0) * _FPW + o, 16)] = px
            ptb[pl.ds((s * 4 + 1) * _FPW + o, 16)] = py
            ptb[pl.ds((s * 4 + 2) * _FPW + o, 16)] = pz
            ptb[pl.ds((s * 4 + 3) * _FPW + o, 16)] = one
        return ()

    lax.fori_loop(0, _CHUNKS, chunk, ())

    for r in range(_NS * 4):
        pltpu.sync_copy(ptb.at[pl.ds(r * _FPW, _FPW)],
                        pt_h.at[pl.ds(r * _FPAD + tb, _FPW)])
    for r in range(8):
        pltpu.sync_copy(wb.at[pl.ds(r * _FPW, _FPW)],
                        w_h.at[pl.ds(r * _FPAD + tb, _FPW)])
    for r in range(6):
        pltpu.sync_copy(citb.at[pl.ds(r * _FPW, _FPW)],
                        cit_h.at[pl.ds(r * _FPAD + tb, _FPW)])
        pltpu.sync_copy(cjb.at[pl.ds(r * _FPW, _FPW)],
                        cj_h.at[pl.ds(r * _FPAD + tb, _FPW)])


_sc_stage1 = functools.partial(
    pl.kernel,
    _sc_body,
    out_type=[
        jax.ShapeDtypeStruct((_NS * 4 * _FPAD,), jnp.float32),
        jax.ShapeDtypeStruct((8 * _FPAD,), jnp.float32),
        jax.ShapeDtypeStruct((6 * _FPAD,), jnp.float32),
        jax.ShapeDtypeStruct((6 * _FPAD,), jnp.float32),
    ],
    mesh=plsc.VectorSubcoreMesh(
        core_axis_name="c", subcore_axis_name="s",
        num_cores=_NC, num_subcores=_NSUB),
    compiler_params=pltpu.CompilerParams(needs_layout_passes=False),
    scratch_types=[
        pltpu.VMEM((2500,), jnp.float32),
        pltpu.VMEM((2500,), jnp.float32),
        pltpu.VMEM((2500,), jnp.float32),
        pltpu.VMEM((_FPW,), jnp.int32),
        pltpu.VMEM((_FPW,), jnp.int32),
        pltpu.VMEM((_FPW,), jnp.int32),
        pltpu.VMEM((3 * _NS * _FPW,), jnp.float32),
        pltpu.VMEM((_NS * 4 * _FPW,), jnp.float32),
        pltpu.VMEM((8 * _FPW,), jnp.float32),
        pltpu.VMEM((6 * _FPW,), jnp.float32),
        pltpu.VMEM((6 * _FPW,), jnp.float32),
    ],
)


def _tc_body(pt_ref, w_ref, cit_ref, cj_ref, fp_ref, out_ref):
    i = pl.program_id(0)
    j = pl.program_id(1)
    dn = (((0,), (0,)), ((), ()))
    hi = lax.Precision.HIGHEST
    d2 = lax.dot_general(cit_ref[...], cj_ref[...], dn, precision=hi)
    ri = lax.broadcasted_iota(jnp.int32, (_I_BLK, _J_BLK), 0) + i * _I_BLK
    rj = lax.broadcasted_iota(jnp.int32, (_I_BLK, _J_BLK), 1) + j * _J_BLK
    mask = (d2 < 1.0) & (ri != rj)
    dds = cj_ref[5:6, :]
    wcat = jnp.concatenate([w_ref[0], w_ref[1]], axis=1)
    has = jnp.zeros((_I_BLK, _J_BLK), jnp.bool_)
    for s in range(_NS):
        r = lax.dot_general(pt_ref[s], wcat, dn, precision=hi)
        nv = r[:, :_J_BLK]
        nw = r[:, _J_BLK:]
        t = (nv + nw) - dds
        has |= (nv >= 0.0) & (nw >= 0.0) & (t <= 0.0)
    partial = jnp.sum(jnp.where(mask & has, fp_ref[...], 0.0))

    @pl.when((i == 0) & (j == 0))
    def _():
        out_ref[0, 0] = 0.0
    acc = out_ref[0, 0] + partial

    @pl.when((i == _NI - 1) & (j == _NJ - 1))
    def _():
        out_ref[0, 0] = acc * (1.0 / 5000.0)

    @pl.when(~((i == _NI - 1) & (j == _NJ - 1)))
    def _():
        out_ref[0, 0] = acc


def _stage2(PT, W, CIT, CJ, fp2):
    out = pl.pallas_call(
        _tc_body,
        grid=(_NI, _NJ),
        in_specs=[
            pl.BlockSpec((_NS, 4, _I_BLK), lambda i, j: (0, 0, i)),
            pl.BlockSpec((2, 4, _J_BLK), lambda i, j: (0, 0, j)),
            pl.BlockSpec((6, _I_BLK), lambda i, j: (0, i)),
            pl.BlockSpec((6, _J_BLK), lambda i, j: (0, j)),
            pl.BlockSpec((_I_BLK, 1), lambda i, j: (i, 0)),
        ],
        out_specs=pl.BlockSpec(memory_space=pltpu.SMEM),
        out_shape=jax.ShapeDtypeStruct((1, 1), jnp.float32),
    )(PT, W, CIT, CJ, fp2)
    return out[0, 0]


@jax.jit
def kernel(vertices, faces, face_probs):
    F = faces.shape[0]
    kk = jax.random.key(42)
    ka, kb = jax.random.split(kk)
    alpha = jax.random.uniform(ka, (F, _NS), dtype=jnp.float32)
    beta = jax.random.uniform(kb, (F, _NS), dtype=jnp.float32) * (1.0 - alpha)
    gamma = 1.0 - alpha - beta
    pad = _FPAD - F
    alf = jnp.pad(alpha, ((0, pad), (0, 0))).T.reshape(-1)
    bef = jnp.pad(beta, ((0, pad), (0, 0))).T.reshape(-1)
    gaf = jnp.pad(gamma, ((0, pad), (0, 0))).T.reshape(-1)
    facesP = jnp.pad(faces, ((0, pad), (0, 0)))
    vx = vertices[:, 0]
    vy = vertices[:, 1]
    vz = vertices[:, 2]
    ptf, wf, citf, cjf = _sc_stage1()(
        vx, vy, vz,
        facesP[:, 0], facesP[:, 1], facesP[:, 2],
        alf, bef, gaf)
    PT = ptf.reshape(_NS, 4, _FPAD)
    W = wf.reshape(2, 4, _FPAD)
    CIT = citf.reshape(6, _FPAD)
    CJ = cjf.reshape(6, _FPAD)
    fp2 = jnp.pad(face_probs, (0, pad)).reshape(_FPAD, 1)
    loss = _stage2(PT, W, CIT, CJ, fp2)
    return loss


# speedup vs baseline: 16.8201x; 1.4167x over previous
import functools
import jax
import jax.numpy as jnp
from jax import lax
from jax.experimental import pallas as pl
from jax.experimental.pallas import tpu as pltpu
from jax.experimental.pallas import tpu_sc as plsc

_NS = 10
_FPAD = 5120
_NC = 2
_NSUB = 16
_NW = _NC * _NSUB
_FPW = _FPAD // _NW
_CHUNKS = _FPW // 16

_I_BLK = 256
_J_BLK = 512
_NI = _FPAD // _I_BLK
_NJ = _FPAD // _J_BLK


_PCW = 32


def _sc_body(vx_h, vy_h, vz_h, f0_h, f1_h, f2_h, al_h, be_h, ga_h,
             pt_h, w_h, cit_h, cj_h, pt2_h,
             vx, vy, vz, f0b, f1b, f2b, alb, ptb, wb, citb, cjb, ptb2):
    wid = lax.axis_index("s") * _NC + lax.axis_index("c")
    tb = wid * _FPW

    pltpu.sync_copy(vx_h, vx)
    pltpu.sync_copy(vy_h, vy)
    pltpu.sync_copy(vz_h, vz)
    pltpu.sync_copy(f0_h.at[pl.ds(tb, _FPW)], f0b)
    pltpu.sync_copy(f1_h.at[pl.ds(tb, _FPW)], f1b)
    pltpu.sync_copy(f2_h.at[pl.ds(tb, _FPW)], f2b)
    for s in range(_NS):
        pltpu.sync_copy(al_h.at[pl.ds(s * _FPAD + tb, _FPW)],
                        alb.at[pl.ds(s * _FPW, _FPW)])
        pltpu.sync_copy(be_h.at[pl.ds(s * _FPAD + tb, _FPW)],
                        alb.at[pl.ds((_NS + s) * _FPW, _FPW)])
        pltpu.sync_copy(ga_h.at[pl.ds(s * _FPAD + tb, _FPW)],
                        alb.at[pl.ds((2 * _NS + s) * _FPW, _FPW)])

    one = jnp.full((16,), 1.0, jnp.float32)
    third = jnp.float32(1.0 / 3.0)

    def chunk(k, _):
        o = k * 16
        f0v = f0b[pl.ds(o, 16)]
        f1v = f1b[pl.ds(o, 16)]
        f2v = f2b[pl.ds(o, 16)]
        v1x = plsc.load_gather(vx, [f0v])
        v1y = plsc.load_gather(vy, [f0v])
        v1z = plsc.load_gather(vz, [f0v])
        v2x = plsc.load_gather(vx, [f1v])
        v2y = plsc.load_gather(vy, [f1v])
        v2z = plsc.load_gather(vz, [f1v])
        v3x = plsc.load_gather(vx, [f2v])
        v3y = plsc.load_gather(vy, [f2v])
        v3z = plsc.load_gather(vz, [f2v])

        e0x = v2x - v1x
        e0y = v2y - v1y
        e0z = v2z - v1z
        e1x = v3x - v1x
        e1y = v3y - v1y
        e1z = v3z - v1z
        d00 = e0x * e0x + e0y * e0y + e0z * e0z
        d01 = e0x * e1x + e0y * e1y + e0z * e1z
        d11 = e1x * e1x + e1y * e1y + e1z * e1z
        den = d00 * d11 - d01 * d01
        c0 = v1x * e0x + v1y * e0y + v1z * e0z
        c1 = v1x * e1x + v1y * e1y + v1z * e1z
        sgn = jnp.where(den > 0.0, 1.0, -1.0).astype(jnp.float32)
        dnz = den != 0.0
        sd00 = sgn * d00
        sd01 = sgn * d01
        sd11 = sgn * d11
        ax = sd11 * e0x - sd01 * e1x
        ay = sd11 * e0y - sd01 * e1y
        az = sd11 * e0z - sd01 * e1z
        a4 = sd01 * c1 - sd11 * c0
        bx = sd00 * e1x - sd01 * e0x
        by = sd00 * e1y - sd01 * e0y
        bz = sd00 * e1z - sd01 * e0z
        b4 = sd01 * c0 - sd00 * c1
        dds = jnp.where(dnz, sgn * den, jnp.float32(-1.0))

        cenx = (v1x + v2x + v3x) * third
        ceny = (v1y + v2y + v3y) * third
        cenz = (v1z + v2z + v3z) * third
        sq = cenx * cenx + ceny * ceny + cenz * cenz
        gid = tb + o + lax.iota(jnp.int32, 16)
        sqm = jnp.where(gid < jnp.int32(5000), sq, jnp.float32(1e9))

        wb[pl.ds(0 * _FPW + o, 16)] = ax
        wb[pl.ds(1 * _FPW + o, 16)] = ay
        wb[pl.ds(2 * _FPW + o, 16)] = az
        wb[pl.ds(3 * _FPW + o, 16)] = a4
        wb[pl.ds(4 * _FPW + o, 16)] = bx
        wb[pl.ds(5 * _FPW + o, 16)] = by
        wb[pl.ds(6 * _FPW + o, 16)] = bz
        wb[pl.ds(7 * _FPW + o, 16)] = b4

        citb[pl.ds(0 * _FPW + o, 16)] = cenx
        citb[pl.ds(1 * _FPW + o, 16)] = ceny
        citb[pl.ds(2 * _FPW + o, 16)] = cenz
        citb[pl.ds(3 * _FPW + o, 16)] = sqm
        citb[pl.ds(4 * _FPW + o, 16)] = one
        citb[pl.ds(5 * _FPW + o, 16)] = jnp.zeros((16,), jnp.float32)

        m2 = jnp.float32(-2.0)
        cjb[pl.ds(0 * _FPW + o, 16)] = m2 * cenx
        cjb[pl.ds(1 * _FPW + o, 16)] = m2 * ceny
        cjb[pl.ds(2 * _FPW + o, 16)] = m2 * cenz
        cjb[pl.ds(3 * _FPW + o, 16)] = one
        cjb[pl.ds(4 * _FPW + o, 16)] = sqm
        cjb[pl.ds(5 * _FPW + o, 16)] = dds

        rowbase = lax.iota(jnp.int32, 16) * _PCW + o * _PCW
        for s in range(_NS):
            al = alb[pl.ds(s * _FPW + o, 16)]
            be = alb[pl.ds((_NS + s) * _FPW + o, 16)]
            ga = alb[pl.ds((2 * _NS + s) * _FPW + o, 16)]
            px = al * v1x + be * v2x + ga * v3x
            py = al * v1y + be * v2y + ga * v3y
            pz = al * v1z + be * v2z + ga * v3z
            ptb[pl.ds((s * 4 + 0) * _FPW + o, 16)] = px
            ptb[pl.ds((s * 4 + 1) * _FPW + o, 16)] = py
            ptb[pl.ds((s * 4 + 2) * _FPW + o, 16)] = pz
            ptb[pl.ds((s * 4 + 3) * _FPW + o, 16)] = one
            plsc.store_scatter(ptb2, [rowbase + (s * 3 + 0)], px)
            plsc.store_scatter(ptb2, [rowbase + (s * 3 + 1)], py)
            plsc.store_scatter(ptb2, [rowbase + (s * 3 + 2)], pz)
        return ()

    lax.fori_loop(0, _CHUNKS, chunk, ())

    for r in range(_NS * 4):
        pltpu.sync_copy(ptb.at[pl.ds(r * _FPW, _FPW)],
                        pt_h.at[pl.ds(r * _FPAD + tb, _FPW)])
    for r in range(8):
        pltpu.sync_copy(wb.at[pl.ds(r * _FPW, _FPW)],
                        w_h.at[pl.ds(r * _FPAD + tb, _FPW)])
    for r in range(6):
        pltpu.sync_copy(citb.at[pl.ds(r * _FPW, _FPW)],
                        cit_h.at[pl.ds(r * _FPAD + tb, _FPW)])
        pltpu.sync_copy(cjb.at[pl.ds(r * _FPW, _FPW)],
                        cj_h.at[pl.ds(r * _FPAD + tb, _FPW)])
    pltpu.sync_copy(ptb2, pt2_h.at[pl.ds(tb * _PCW, _FPW * _PCW)])


_sc_stage1 = functools.partial(
    pl.kernel,
    _sc_body,
    out_type=[
        jax.ShapeDtypeStruct((_NS * 4 * _FPAD,), jnp.float32),
        jax.ShapeDtypeStruct((8 * _FPAD,), jnp.float32),
        jax.ShapeDtypeStruct((6 * _FPAD,), jnp.float32),
        jax.ShapeDtypeStruct((6 * _FPAD,), jnp.float32),
        jax.ShapeDtypeStruct((_FPAD * _PCW,), jnp.float32),
    ],
    mesh=plsc.VectorSubcoreMesh(
        core_axis_name="c", subcore_axis_name="s",
        num_cores=_NC, num_subcores=_NSUB),
    compiler_params=pltpu.CompilerParams(needs_layout_passes=False),
    scratch_types=[
        pltpu.VMEM((2500,), jnp.float32),
        pltpu.VMEM((2500,), jnp.float32),
        pltpu.VMEM((2500,), jnp.float32),
        pltpu.VMEM((_FPW,), jnp.int32),
        pltpu.VMEM((_FPW,), jnp.int32),
        pltpu.VMEM((_FPW,), jnp.int32),
        pltpu.VMEM((3 * _NS * _FPW,), jnp.float32),
        pltpu.VMEM((_NS * 4 * _FPW,), jnp.float32),
        pltpu.VMEM((8 * _FPW,), jnp.float32),
        pltpu.VMEM((6 * _FPW,), jnp.float32),
        pltpu.VMEM((6 * _FPW,), jnp.float32),
        pltpu.VMEM((_FPW * _PCW,), jnp.float32),
    ],
)


_MXU_S = 5


def _tc_body(pt_ref, pc_ref, w_ref, cit_ref, cj_ref, fp_ref, out_ref):
    i = pl.program_id(0)
    j = pl.program_id(1)
    dn = (((0,), (0,)), ((), ()))
    hi = lax.Precision.HIGHEST
    d2 = lax.dot_general(cit_ref[...], cj_ref[...], dn, precision=hi)
    ri = lax.broadcasted_iota(jnp.int32, (_I_BLK, _J_BLK), 0) + i * _I_BLK
    rj = lax.broadcasted_iota(jnp.int32, (_I_BLK, _J_BLK), 1) + j * _J_BLK
    mask = (d2 < 1.0) & (ri != rj)
    dds = cj_ref[5:6, :]
    wcat = jnp.concatenate([w_ref[0], w_ref[1]], axis=1)
    has = jnp.zeros((_I_BLK, _J_BLK), jnp.bool_)
    for s in range(_MXU_S):
        r = lax.dot_general(pt_ref[s], wcat, dn, precision=hi)
        nv = r[:, :_J_BLK]
        nw = r[:, _J_BLK:]
        t = (nv + nw) - dds
        has |= (nv >= 0.0) & (nw >= 0.0) & (t <= 0.0)
    ax = w_ref[0, 0:1, :]
    ay = w_ref[0, 1:2, :]
    az = w_ref[0, 2:3, :]
    a4 = w_ref[0, 3:4, :]
    bx = w_ref[1, 0:1, :]
    by = w_ref[1, 1:2, :]
    bz = w_ref[1, 2:3, :]
    b4 = w_ref[1, 3:4, :]
    for s in range(_MXU_S, _NS):
        px = pc_ref[:, (s * 3 + 0):(s * 3 + 1)]
        py = pc_ref[:, (s * 3 + 1):(s * 3 + 2)]
        pz = pc_ref[:, (s * 3 + 2):(s * 3 + 3)]
        nv = px * ax + py * ay + pz * az + a4
        nw = px * bx + py * by + pz * bz + b4
        t = (nv + nw) - dds
        has |= (nv >= 0.0) & (nw >= 0.0) & (t <= 0.0)
    partial = jnp.sum(jnp.where(mask & has, fp_ref[...], 0.0))

    @pl.when((i == 0) & (j == 0))
    def _():
        out_ref[0, 0] = 0.0
    acc = out_ref[0, 0] + partial

    @pl.when((i == _NI - 1) & (j == _NJ - 1))
    def _():
        out_ref[0, 0] = acc * (1.0 / 5000.0)

    @pl.when(~((i == _NI - 1) & (j == _NJ - 1)))
    def _():
        out_ref[0, 0] = acc


def _stage2(PT, PC, W, CIT, CJ, fp2):
    out = pl.pallas_call(
        _tc_body,
        grid=(_NI, _NJ),
        in_specs=[
            pl.BlockSpec((_NS, 4, _I_BLK), lambda i, j: (0, 0, i)),
            pl.BlockSpec((_I_BLK, _PCW), lambda i, j: (i, 0)),
            pl.BlockSpec((2, 4, _J_BLK), lambda i, j: (0, 0, j)),
            pl.BlockSpec((6, _I_BLK), lambda i, j: (0, i)),
            pl.BlockSpec((6, _J_BLK), lambda i, j: (0, j)),
            pl.BlockSpec((_I_BLK, 1), lambda i, j: (i, 0)),
        ],
        out_specs=pl.BlockSpec(memory_space=pltpu.SMEM),
        out_shape=jax.ShapeDtypeStruct((1, 1), jnp.float32),
    )(PT, PC, W, CIT, CJ, fp2)
    return out[0, 0]


@jax.jit
def kernel(vertices, faces, face_probs):
    F = faces.shape[0]
    kk = jax.random.key(42)
    ka, kb = jax.random.split(kk)
    alpha = jax.random.uniform(ka, (F, _NS), dtype=jnp.float32)
    beta = jax.random.uniform(kb, (F, _NS), dtype=jnp.float32) * (1.0 - alpha)
    gamma = 1.0 - alpha - beta
    pad = _FPAD - F
    alf = jnp.pad(alpha, ((0, pad), (0, 0))).T.reshape(-1)
    bef = jnp.pad(beta, ((0, pad), (0, 0))).T.reshape(-1)
    gaf = jnp.pad(gamma, ((0, pad), (0, 0))).T.reshape(-1)
    facesP = jnp.pad(faces, ((0, pad), (0, 0)))
    vx = vertices[:, 0]
    vy = vertices[:, 1]
    vz = vertices[:, 2]
    ptf, wf, citf, cjf, pt2f = _sc_stage1()(
        vx, vy, vz,
        facesP[:, 0], facesP[:, 1], facesP[:, 2],
        alf, bef, gaf)
    PT = ptf.reshape(_NS, 4, _FPAD)
    PC = pt2f.reshape(_FPAD, _PCW)
    W = wf.reshape(2, 4, _FPAD)
    CIT = citf.reshape(6, _FPAD)
    CJ = cjf.reshape(6, _FPAD)
    fp2 = jnp.pad(face_probs, (0, pad)).reshape(_FPAD, 1)
    loss = _stage2(PT, PC, W, CIT, CJ, fp2)
    return loss


# speedup vs baseline: 40.8771x; 2.4303x over previous
import functools
import jax
import jax.numpy as jnp
from jax import lax
from jax.experimental import pallas as pl
from jax.experimental.pallas import tpu as pltpu
from jax.experimental.pallas import tpu_sc as plsc

_NS = 10
_FPAD = 5120
_NV = 2500
_NC = 2
_NSUB = 16
_NW = _NC * _NSUB
_FPW = _FPAD // _NW
_CHUNKS = _FPW // 16
_PCW = 32

_I_BLK = 256
_J_BLK = 512
_NI = _FPAD // _I_BLK
_NJ = _FPAD // _J_BLK

_MXU_S = 5

_SC_MESH = plsc.VectorSubcoreMesh(
    core_axis_name="c", subcore_axis_name="s",
    num_cores=_NC, num_subcores=_NSUB)
_SC_PARAMS = pltpu.CompilerParams(needs_layout_passes=False)


def _keys_body(vx_h, f0_h, f1_h, f2_h, keys_h, vx, f0b, f1b, f2b, keysb):
    wid = lax.axis_index("s") * _NC + lax.axis_index("c")
    tb = wid * _FPW
    pltpu.sync_copy(vx_h, vx)
    pltpu.sync_copy(f0_h.at[pl.ds(tb, _FPW)], f0b)
    pltpu.sync_copy(f1_h.at[pl.ds(tb, _FPW)], f1b)
    pltpu.sync_copy(f2_h.at[pl.ds(tb, _FPW)], f2b)
    third = jnp.float32(1.0 / 3.0)

    def chunk(k, _):
        o = k * 16
        v1x = plsc.load_gather(vx, [f0b[pl.ds(o, 16)]])
        v2x = plsc.load_gather(vx, [f1b[pl.ds(o, 16)]])
        v3x = plsc.load_gather(vx, [f2b[pl.ds(o, 16)]])
        cx = (v1x + v2x + v3x) * third
        gid = tb + o + lax.iota(jnp.int32, 16)
        keysb[pl.ds(o, 16)] = jnp.where(gid < jnp.int32(5000), cx,
                                        jnp.float32(1e9))
        return ()

    lax.fori_loop(0, _CHUNKS, chunk, ())
    pltpu.sync_copy(keysb, keys_h.at[pl.ds(tb, _FPW)])


_sc_keys = functools.partial(
    pl.kernel,
    _keys_body,
    out_type=[jax.ShapeDtypeStruct((_FPAD,), jnp.float32)],
    mesh=_SC_MESH,
    compiler_params=_SC_PARAMS,
    scratch_types=[
        pltpu.VMEM((_NV,), jnp.float32),
        pltpu.VMEM((_FPW,), jnp.int32),
        pltpu.VMEM((_FPW,), jnp.int32),
        pltpu.VMEM((_FPW,), jnp.int32),
        pltpu.VMEM((_FPW,), jnp.float32),
    ],
)


def _sc_body(vx_h, vy_h, vz_h, f0_h, f1_h, f2_h, fp_h, abg_h, perm_h,
             pt_h, w_h, cit_h, cj_h, pt2_h, fps_h,
             vx, vy, vz, f0f, f1f, f2f, fpf, permb, abgb,
             ptb, wb, citb, cjb, ptb2, fpsb, sem):
    wid = lax.axis_index("s") * _NC + lax.axis_index("c")
    tb = wid * _FPW

    pltpu.sync_copy(vx_h, vx)
    pltpu.sync_copy(vy_h, vy)
    pltpu.sync_copy(vz_h, vz)
    pltpu.sync_copy(f0_h, f0f)
    pltpu.sync_copy(f1_h, f1f)
    pltpu.sync_copy(f2_h, f2f)
    pltpu.sync_copy(fp_h, fpf)
    pltpu.sync_copy(perm_h.at[pl.ds(tb, _FPW)], permb)
    copies = []
    for k in range(_CHUNKS):
        idxv = permb[pl.ds(k * 16, 16)]
        copies.append(pltpu.async_copy(
            abg_h.at[idxv], abgb.at[pl.ds(k * 16, 16)], sem))
    for c in copies:
        c.wait()

    one = jnp.full((16,), 1.0, jnp.float32)
    third = jnp.float32(1.0 / 3.0)

    def chunk(k, _):
        o = k * 16
        rows = o + lax.iota(jnp.int32, 16)
        pch = permb[pl.ds(o, 16)]
        f0v = plsc.load_gather(f0f, [pch])
        f1v = plsc.load_gather(f1f, [pch])
        f2v = plsc.load_gather(f2f, [pch])
        fpv = plsc.load_gather(fpf, [pch])
        v1x = plsc.load_gather(vx, [f0v])
        v1y = plsc.load_gather(vy, [f0v])
        v1z = plsc.load_gather(vz, [f0v])
        v2x = plsc.load_gather(vx, [f1v])
        v2y = plsc.load_gather(vy, [f1v])
        v2z = plsc.load_gather(vz, [f1v])
        v3x = plsc.load_gather(vx, [f2v])
        v3y = plsc.load_gather(vy, [f2v])
        v3z = plsc.load_gather(vz, [f2v])

        e0x = v2x - v1x
        e0y = v2y - v1y
        e0z = v2z - v1z
        e1x = v3x - v1x
        e1y = v3y - v1y
        e1z = v3z - v1z
        d00 = e0x * e0x + e0y * e0y + e0z * e0z
        d01 = e0x * e1x + e0y * e1y + e0z * e1z
        d11 = e1x * e1x + e1y * e1y + e1z * e1z
        den = d00 * d11 - d01 * d01
        c0 = v1x * e0x + v1y * e0y + v1z * e0z
        c1 = v1x * e1x + v1y * e1y + v1z * e1z
        sgn = jnp.where(den > 0.0, 1.0, -1.0).astype(jnp.float32)
        dnz = den != 0.0
        sd00 = sgn * d00
        sd01 = sgn * d01
        sd11 = sgn * d11
        ax = sd11 * e0x - sd01 * e1x
        ay = sd11 * e0y - sd01 * e1y
        az = sd11 * e0z - sd01 * e1z
        a4 = sd01 * c1 - sd11 * c0
        bx = sd00 * e1x - sd01 * e0x
        by = sd00 * e1y - sd01 * e0y
        bz = sd00 * e1z - sd01 * e0z
        b4 = sd01 * c0 - sd00 * c1
        dds = jnp.where(dnz, sgn * den, jnp.float32(-1.0))

        cenx = (v1x + v2x + v3x) * third
        ceny = (v1y + v2y + v3y) * third
        cenz = (v1z + v2z + v3z) * third
        sq = cenx * cenx + ceny * ceny + cenz * cenz
        sqm = jnp.where(pch < jnp.int32(5000), sq, jnp.float32(1e9))

        wb[pl.ds(0 * _FPW + o, 16)] = ax
        wb[pl.ds(1 * _FPW + o, 16)] = ay
        wb[pl.ds(2 * _FPW + o, 16)] = az
        wb[pl.ds(3 * _FPW + o, 16)] = a4
        wb[pl.ds(4 * _FPW + o, 16)] = bx
        wb[pl.ds(5 * _FPW + o, 16)] = by
        wb[pl.ds(6 * _FPW + o, 16)] = bz
        wb[pl.ds(7 * _FPW + o, 16)] = b4

        citb[pl.ds(0 * _FPW + o, 16)] = cenx
        citb[pl.ds(1 * _FPW + o, 16)] = ceny
        citb[pl.ds(2 * _FPW + o, 16)] = cenz
        citb[pl.ds(3 * _FPW + o, 16)] = sqm
        citb[pl.ds(4 * _FPW + o, 16)] = one
        citb[pl.ds(5 * _FPW + o, 16)] = jnp.zeros((16,), jnp.float32)

        m2 = jnp.float32(-2.0)
        cjb[pl.ds(0 * _FPW + o, 16)] = m2 * cenx
        cjb[pl.ds(1 * _FPW + o, 16)] = m2 * ceny
        cjb[pl.ds(2 * _FPW + o, 16)] = m2 * cenz
        cjb[pl.ds(3 * _FPW + o, 16)] = one
        cjb[pl.ds(4 * _FPW + o, 16)] = sqm
        cjb[pl.ds(5 * _FPW + o, 16)] = dds

        fpsb[pl.ds(o, 16)] = fpv

        rowbase = lax.iota(jnp.int32, 16) * _PCW + o * _PCW
        for s in range(_NS):
            colal = jnp.full((16,), s, jnp.int32)
            colbe = jnp.full((16,), _NS + s, jnp.int32)
            colga = jnp.full((16,), 2 * _NS + s, jnp.int32)
            al = plsc.load_gather(abgb, [rows, colal])
            be = plsc.load_gather(abgb, [rows, colbe])
            ga = plsc.load_gather(abgb, [rows, colga])
            px = al * v1x + be * v2x + ga * v3x
            py = al * v1y + be * v2y + ga * v3y
            pz = al * v1z + be * v2z + ga * v3z
            ptb[pl.ds((s * 4 + 0) * _FPW + o, 16)] = px
            ptb[pl.ds((s * 4 + 1) * _FPW + o, 16)] = py
            ptb[pl.ds((s * 4 + 2) * _FPW + o, 16)] = pz
            ptb[pl.ds((s * 4 + 3) * _FPW + o, 16)] = one
            plsc.store_scatter(ptb2, [rowbase + (s * 3 + 0)], px)
            plsc.store_scatter(ptb2, [rowbase + (s * 3 + 1)], py)
            plsc.store_scatter(ptb2, [rowbase + (s * 3 + 2)], pz)
        return ()

    lax.fori_loop(0, _CHUNKS, chunk, ())

    for r in range(_NS * 4):
        pltpu.sync_copy(ptb.at[pl.ds(r * _FPW, _FPW)],
                        pt_h.at[pl.ds(r * _FPAD + tb, _FPW)])
    for r in range(8):
        pltpu.sync_copy(wb.at[pl.ds(r * _FPW, _FPW)],
                        w_h.at[pl.ds(r * _FPAD + tb, _FPW)])
    for r in range(6):
        pltpu.sync_copy(citb.at[pl.ds(r * _FPW, _FPW)],
                        cit_h.at[pl.ds(r * _FPAD + tb, _FPW)])
        pltpu.sync_copy(cjb.at[pl.ds(r * _FPW, _FPW)],
                        cj_h.at[pl.ds(r * _FPAD + tb, _FPW)])
    pltpu.sync_copy(ptb2, pt2_h.at[pl.ds(tb * _PCW, _FPW * _PCW)])
    pltpu.sync_copy(fpsb, fps_h.at[pl.ds(tb, _FPW)])


_sc_stage1 = functools.partial(
    pl.kernel,
    _sc_body,
    out_type=[
        jax.ShapeDtypeStruct((_NS * 4 * _FPAD,), jnp.float32),
        jax.ShapeDtypeStruct((8 * _FPAD,), jnp.float32),
        jax.ShapeDtypeStruct((6 * _FPAD,), jnp.float32),
        jax.ShapeDtypeStruct((6 * _FPAD,), jnp.float32),
        jax.ShapeDtypeStruct((_FPAD * _PCW,), jnp.float32),
        jax.ShapeDtypeStruct((_FPAD,), jnp.float32),
    ],
    mesh=_SC_MESH,
    compiler_params=_SC_PARAMS,
    scratch_types=[
        pltpu.VMEM((_NV,), jnp.float32),
        pltpu.VMEM((_NV,), jnp.float32),
        pltpu.VMEM((_NV,), jnp.float32),
        pltpu.VMEM((_FPAD,), jnp.int32),
        pltpu.VMEM((_FPAD,), jnp.int32),
        pltpu.VMEM((_FPAD,), jnp.int32),
        pltpu.VMEM((_FPAD,), jnp.float32),
        pltpu.VMEM((_FPW,), jnp.int32),
        pltpu.VMEM((_FPW, 128), jnp.float32),
        pltpu.VMEM((_NS * 4 * _FPW,), jnp.float32),
        pltpu.VMEM((8 * _FPW,), jnp.float32),
        pltpu.VMEM((6 * _FPW,), jnp.float32),
        pltpu.VMEM((6 * _FPW,), jnp.float32),
        pltpu.VMEM((_FPW * _PCW,), jnp.float32),
        pltpu.VMEM((_FPW,), jnp.float32),
        pltpu.SemaphoreType.DMA,
    ],
)


def _tc_body(flags_ref, pt_ref, pc_ref, w_ref, cit_ref, cj_ref, fp_ref,
             out_ref):
    i = pl.program_id(0)
    j = pl.program_id(1)

    @pl.when((i == 0) & (j == 0))
    def _():
        out_ref[0, 0] = 0.0

    @pl.when(flags_ref[i, j] != 0)
    def _():
        dn = (((0,), (0,)), ((), ()))
        hi = lax.Precision.HIGHEST
        d2 = lax.dot_general(cit_ref[...], cj_ref[...], dn, precision=hi)
        ri = lax.broadcasted_iota(jnp.int32, (_I_BLK, _J_BLK), 0) + i * _I_BLK
        rj = lax.broadcasted_iota(jnp.int32, (_I_BLK, _J_BLK), 1) + j * _J_BLK
        mask = (d2 < 1.0) & (ri != rj)
        dds = cj_ref[5:6, :]
        wcat = jnp.concatenate([w_ref[0], w_ref[1]], axis=1)
        has = jnp.zeros((_I_BLK, _J_BLK), jnp.bool_)
        for s in range(_MXU_S):
            r = lax.dot_general(pt_ref[s], wcat, dn, precision=hi)
            nv = r[:, :_J_BLK]
            nw = r[:, _J_BLK:]
            t = (nv + nw) - dds
            has |= (nv >= 0.0) & (nw >= 0.0) & (t <= 0.0)
        ax = w_ref[0, 0:1, :]
        ay = w_ref[0, 1:2, :]
        az = w_ref[0, 2:3, :]
        a4 = w_ref[0, 3:4, :]
        bx = w_ref[1, 0:1, :]
        by = w_ref[1, 1:2, :]
        bz = w_ref[1, 2:3, :]
        b4 = w_ref[1, 3:4, :]
        for s in range(_MXU_S, _NS):
            px = pc_ref[:, (s * 3 + 0):(s * 3 + 1)]
            py = pc_ref[:, (s * 3 + 1):(s * 3 + 2)]
            pz = pc_ref[:, (s * 3 + 2):(s * 3 + 3)]
            nv = px * ax + py * ay + pz * az + a4
            nw = px * bx + py * by + pz * bz + b4
            t = (nv + nw) - dds
            has |= (nv >= 0.0) & (nw >= 0.0) & (t <= 0.0)
        partial = jnp.sum(jnp.where(mask & has, fp_ref[...], 0.0))
        out_ref[0, 0] = out_ref[0, 0] + partial

    @pl.when((i == _NI - 1) & (j == _NJ - 1))
    def _():
        out_ref[0, 0] = out_ref[0, 0] * (1.0 / 5000.0)


def _stage2(flags, PT, PC, W, CIT, CJ, fp2):
    out = pl.pallas_call(
        _tc_body,
        grid=(_NI, _NJ),
        in_specs=[
            pl.BlockSpec(memory_space=pltpu.SMEM),
            pl.BlockSpec((_NS, 4, _I_BLK), lambda i, j: (0, 0, i)),
            pl.BlockSpec((_I_BLK, _PCW), lambda i, j: (i, 0)),
            pl.BlockSpec((2, 4, _J_BLK), lambda i, j: (0, 0, j)),
            pl.BlockSpec((6, _I_BLK), lambda i, j: (0, i)),
            pl.BlockSpec((6, _J_BLK), lambda i, j: (0, j)),
            pl.BlockSpec((_I_BLK, 1), lambda i, j: (i, 0)),
        ],
        out_specs=pl.BlockSpec(memory_space=pltpu.SMEM),
        out_shape=jax.ShapeDtypeStruct((1, 1), jnp.float32),
    )(flags, PT, PC, W, CIT, CJ, fp2)
    return out[0, 0]


@jax.jit
def kernel(vertices, faces, face_probs):
    F = faces.shape[0]
    kk = jax.random.key(42)
    ka, kb = jax.random.split(kk)
    alpha = jax.random.uniform(ka, (F, _NS), dtype=jnp.float32)
    beta = jax.random.uniform(kb, (F, _NS), dtype=jnp.float32) * (1.0 - alpha)
    gamma = 1.0 - alpha - beta
    pad = _FPAD - F
    abg = jnp.pad(jnp.concatenate([alpha, beta, gamma], axis=1),
                  ((0, pad), (0, 128 - 3 * _NS)))
    facesP = jnp.pad(faces, ((0, pad), (0, 0)))
    f0 = facesP[:, 0]
    f1 = facesP[:, 1]
    f2 = facesP[:, 2]
    vx = vertices[:, 0]
    vy = vertices[:, 1]
    vz = vertices[:, 2]
    fpP = jnp.pad(face_probs, (0, pad))

    (keys,) = _sc_keys()(vx, f0, f1, f2)
    perm = jnp.argsort(keys).astype(jnp.int32)
    xs = jnp.sort(keys)
    xlo_i = xs[::_I_BLK]
    xhi_i = xs[_I_BLK - 1::_I_BLK]
    xlo_j = xs[::_J_BLK]
    xhi_j = xs[_J_BLK - 1::_J_BLK]
    eps = jnp.float32(1e-3)
    flags = ((xlo_j[None, :] <= xhi_i[:, None] + 1.0 + eps)
             & (xlo_i[:, None] <= xhi_j[None, :] + 1.0 + eps)).astype(jnp.int32)

    ptf, wf, citf, cjf, pt2f, fps = _sc_stage1()(
        vx, vy, vz, f0, f1, f2, fpP, abg, perm)
    PT = ptf.reshape(_NS, 4, _FPAD)
    PC = pt2f.reshape(_FPAD, _PCW)
    W = wf.reshape(2, 4, _FPAD)
    CIT = citf.reshape(6, _FPAD)
    CJ = cjf.reshape(6, _FPAD)
    fp2 = fps.reshape(_FPAD, 1)
    loss = _stage2(flags, PT, PC, W, CIT, CJ, fp2)
    return loss


# speedup vs baseline: 43.3044x; 1.0594x over previous
import functools
import jax
import jax.numpy as jnp
from jax import lax
from jax.experimental import pallas as pl
from jax.experimental.pallas import tpu as pltpu
from jax.experimental.pallas import tpu_sc as plsc

_NS = 10
_FPAD = 5120
_NV = 2500
_NC = 2
_NSUB = 16
_NW = _NC * _NSUB
_FPW = _FPAD // _NW
_CHUNKS = _FPW // 16
_PCW = 32

_I_BLK = 256
_J_BLK = 512
_NI = _FPAD // _I_BLK
_NJ = _FPAD // _J_BLK

_MXU_S = 5

_SC_MESH = plsc.VectorSubcoreMesh(
    core_axis_name="c", subcore_axis_name="s",
    num_cores=_NC, num_subcores=_NSUB)
_SC_PARAMS = pltpu.CompilerParams(needs_layout_passes=False)


def _keys_body(vx_h, f0_h, f1_h, f2_h, keys_h, vx, f0b, f1b, f2b, keysb):
    wid = lax.axis_index("s") * _NC + lax.axis_index("c")
    tb = wid * _FPW
    pltpu.sync_copy(vx_h, vx)
    pltpu.sync_copy(f0_h.at[pl.ds(tb, _FPW)], f0b)
    pltpu.sync_copy(f1_h.at[pl.ds(tb, _FPW)], f1b)
    pltpu.sync_copy(f2_h.at[pl.ds(tb, _FPW)], f2b)
    third = jnp.float32(1.0 / 3.0)

    def chunk(k, _):
        o = k * 16
        v1x = plsc.load_gather(vx, [f0b[pl.ds(o, 16)]])
        v2x = plsc.load_gather(vx, [f1b[pl.ds(o, 16)]])
        v3x = plsc.load_gather(vx, [f2b[pl.ds(o, 16)]])
        cx = (v1x + v2x + v3x) * third
        gid = tb + o + lax.iota(jnp.int32, 16)
        keysb[pl.ds(o, 16)] = jnp.where(gid < jnp.int32(5000), cx,
                                        jnp.float32(1e9))
        return ()

    lax.fori_loop(0, _CHUNKS, chunk, ())
    pltpu.sync_copy(keysb, keys_h.at[pl.ds(tb, _FPW)])


_sc_keys = functools.partial(
    pl.kernel,
    _keys_body,
    out_type=[jax.ShapeDtypeStruct((_FPAD,), jnp.float32)],
    mesh=_SC_MESH,
    compiler_params=_SC_PARAMS,
    scratch_types=[
        pltpu.VMEM((_NV,), jnp.float32),
        pltpu.VMEM((_FPW,), jnp.int32),
        pltpu.VMEM((_FPW,), jnp.int32),
        pltpu.VMEM((_FPW,), jnp.int32),
        pltpu.VMEM((_FPW,), jnp.float32),
    ],
)


def _sc_body(vx_h, vy_h, vz_h, f0_h, f1_h, f2_h, fp_h, abg_h, perm_h,
             pt_h, w_h, cit_h, cj_h, pt2_h, fps_h,
             vx, vy, vz, f0f, f1f, f2f, fpf, permb, abgb,
             ptb, wb, citb, cjb, ptb2, fpsb, sem):
    wid = lax.axis_index("s") * _NC + lax.axis_index("c")
    tb = wid * _FPW

    pltpu.sync_copy(vx_h, vx)
    pltpu.sync_copy(vy_h, vy)
    pltpu.sync_copy(vz_h, vz)
    pltpu.sync_copy(f0_h, f0f)
    pltpu.sync_copy(f1_h, f1f)
    pltpu.sync_copy(f2_h, f2f)
    pltpu.sync_copy(fp_h, fpf)
    pltpu.sync_copy(perm_h.at[pl.ds(tb, _FPW)], permb)
    copies = []
    for k in range(_CHUNKS):
        idxv = permb[pl.ds(k * 16, 16)]
        copies.append(pltpu.async_copy(
            abg_h.at[idxv], abgb.at[pl.ds(k * 16, 16)], sem))
    for c in copies:
        c.wait()

    one = jnp.full((16,), 1.0, jnp.float32)
    third = jnp.float32(1.0 / 3.0)

    def chunk(k, _):
        o = k * 16
        rows = o + lax.iota(jnp.int32, 16)
        pch = permb[pl.ds(o, 16)]
        f0v = plsc.load_gather(f0f, [pch])
        f1v = plsc.load_gather(f1f, [pch])
        f2v = plsc.load_gather(f2f, [pch])
        fpv = plsc.load_gather(fpf, [pch])
        v1x = plsc.load_gather(vx, [f0v])
        v1y = plsc.load_gather(vy, [f0v])
        v1z = plsc.load_gather(vz, [f0v])
        v2x = plsc.load_gather(vx, [f1v])
        v2y = plsc.load_gather(vy, [f1v])
        v2z = plsc.load_gather(vz, [f1v])
        v3x = plsc.load_gather(vx, [f2v])
        v3y = plsc.load_gather(vy, [f2v])
        v3z = plsc.load_gather(vz, [f2v])

        e0x = v2x - v1x
        e0y = v2y - v1y
        e0z = v2z - v1z
        e1x = v3x - v1x
        e1y = v3y - v1y
        e1z = v3z - v1z
        d00 = e0x * e0x + e0y * e0y + e0z * e0z
        d01 = e0x * e1x + e0y * e1y + e0z * e1z
        d11 = e1x * e1x + e1y * e1y + e1z * e1z
        den = d00 * d11 - d01 * d01
        c0 = v1x * e0x + v1y * e0y + v1z * e0z
        c1 = v1x * e1x + v1y * e1y + v1z * e1z
        sgn = jnp.where(den > 0.0, 1.0, -1.0).astype(jnp.float32)
        dnz = den != 0.0
        sd00 = sgn * d00
        sd01 = sgn * d01
        sd11 = sgn * d11
        ax = sd11 * e0x - sd01 * e1x
        ay = sd11 * e0y - sd01 * e1y
        az = sd11 * e0z - sd01 * e1z
        a4 = sd01 * c1 - sd11 * c0
        bx = sd00 * e1x - sd01 * e0x
        by = sd00 * e1y - sd01 * e0y
        bz = sd00 * e1z - sd01 * e0z
        b4 = sd01 * c0 - sd00 * c1
        dds = jnp.where(dnz, sgn * den, jnp.float32(-1.0))

        cenx = (v1x + v2x + v3x) * third
        ceny = (v1y + v2y + v3y) * third
        cenz = (v1z + v2z + v3z) * third
        sq = cenx * cenx + ceny * ceny + cenz * cenz
        sqm = jnp.where(pch < jnp.int32(5000), sq, jnp.float32(1e9))

        wb[pl.ds(0 * _FPW + o, 16)] = ax
        wb[pl.ds(1 * _FPW + o, 16)] = ay
        wb[pl.ds(2 * _FPW + o, 16)] = az
        wb[pl.ds(3 * _FPW + o, 16)] = a4
        wb[pl.ds(4 * _FPW + o, 16)] = bx
        wb[pl.ds(5 * _FPW + o, 16)] = by
        wb[pl.ds(6 * _FPW + o, 16)] = bz
        wb[pl.ds(7 * _FPW + o, 16)] = b4

        citb[pl.ds(0 * _FPW + o, 16)] = cenx
        citb[pl.ds(1 * _FPW + o, 16)] = ceny
        citb[pl.ds(2 * _FPW + o, 16)] = cenz
        citb[pl.ds(3 * _FPW + o, 16)] = sqm
        citb[pl.ds(4 * _FPW + o, 16)] = one
        citb[pl.ds(5 * _FPW + o, 16)] = jnp.zeros((16,), jnp.float32)

        m2 = jnp.float32(-2.0)
        cjb[pl.ds(0 * _FPW + o, 16)] = m2 * cenx
        cjb[pl.ds(1 * _FPW + o, 16)] = m2 * ceny
        cjb[pl.ds(2 * _FPW + o, 16)] = m2 * cenz
        cjb[pl.ds(3 * _FPW + o, 16)] = one
        cjb[pl.ds(4 * _FPW + o, 16)] = sqm
        cjb[pl.ds(5 * _FPW + o, 16)] = dds

        fpsb[pl.ds(o, 16)] = fpv

        rowbase = lax.iota(jnp.int32, 16) * _PCW + o * _PCW
        for s in range(_NS):
            colal = jnp.full((16,), s, jnp.int32)
            colbe = jnp.full((16,), _NS + s, jnp.int32)
            colga = jnp.full((16,), 2 * _NS + s, jnp.int32)
            al = plsc.load_gather(abgb, [rows, colal])
            be = plsc.load_gather(abgb, [rows, colbe])
            ga = plsc.load_gather(abgb, [rows, colga])
            px = al * v1x + be * v2x + ga * v3x
            py = al * v1y + be * v2y + ga * v3y
            pz = al * v1z + be * v2z + ga * v3z
            ptb[pl.ds((s * 4 + 0) * _FPW + o, 16)] = px
            ptb[pl.ds((s * 4 + 1) * _FPW + o, 16)] = py
            ptb[pl.ds((s * 4 + 2) * _FPW + o, 16)] = pz
            ptb[pl.ds((s * 4 + 3) * _FPW + o, 16)] = one
            plsc.store_scatter(ptb2, [rowbase + (s * 3 + 0)], px)
            plsc.store_scatter(ptb2, [rowbase + (s * 3 + 1)], py)
            plsc.store_scatter(ptb2, [rowbase + (s * 3 + 2)], pz)
        return ()

    lax.fori_loop(0, _CHUNKS, chunk, ())

    for r in range(_NS * 4):
        pltpu.sync_copy(ptb.at[pl.ds(r * _FPW, _FPW)],
                        pt_h.at[pl.ds(r * _FPAD + tb, _FPW)])
    for r in range(8):
        pltpu.sync_copy(wb.at[pl.ds(r * _FPW, _FPW)],
                        w_h.at[pl.ds(r * _FPAD + tb, _FPW)])
    for r in range(6):
        pltpu.sync_copy(citb.at[pl.ds(r * _FPW, _FPW)],
                        cit_h.at[pl.ds(r * _FPAD + tb, _FPW)])
        pltpu.sync_copy(cjb.at[pl.ds(r * _FPW, _FPW)],
                        cj_h.at[pl.ds(r * _FPAD + tb, _FPW)])
    pltpu.sync_copy(ptb2, pt2_h.at[pl.ds(tb * _PCW, _FPW * _PCW)])
    pltpu.sync_copy(fpsb, fps_h.at[pl.ds(tb, _FPW)])


_sc_stage1 = functools.partial(
    pl.kernel,
    _sc_body,
    out_type=[
        jax.ShapeDtypeStruct((_NS * 4 * _FPAD,), jnp.float32),
        jax.ShapeDtypeStruct((8 * _FPAD,), jnp.float32),
        jax.ShapeDtypeStruct((6 * _FPAD,), jnp.float32),
        jax.ShapeDtypeStruct((6 * _FPAD,), jnp.float32),
        jax.ShapeDtypeStruct((_FPAD * _PCW,), jnp.float32),
        jax.ShapeDtypeStruct((_FPAD,), jnp.float32),
    ],
    mesh=_SC_MESH,
    compiler_params=_SC_PARAMS,
    scratch_types=[
        pltpu.VMEM((_NV,), jnp.float32),
        pltpu.VMEM((_NV,), jnp.float32),
        pltpu.VMEM((_NV,), jnp.float32),
        pltpu.VMEM((_FPAD,), jnp.int32),
        pltpu.VMEM((_FPAD,), jnp.int32),
        pltpu.VMEM((_FPAD,), jnp.int32),
        pltpu.VMEM((_FPAD,), jnp.float32),
        pltpu.VMEM((_FPW,), jnp.int32),
        pltpu.VMEM((_FPW, 128), jnp.float32),
        pltpu.VMEM((_NS * 4 * _FPW,), jnp.float32),
        pltpu.VMEM((8 * _FPW,), jnp.float32),
        pltpu.VMEM((6 * _FPW,), jnp.float32),
        pltpu.VMEM((6 * _FPW,), jnp.float32),
        pltpu.VMEM((_FPW * _PCW,), jnp.float32),
        pltpu.VMEM((_FPW,), jnp.float32),
        pltpu.SemaphoreType.DMA,
    ],
)


def _tc_body(flags_ref, pt_ref, pc_ref, w_ref, cit_ref, cj_ref, fp_ref,
             out_ref):
    i = pl.program_id(0)
    j = pl.program_id(1)

    @pl.when((i == 0) & (j == 0))
    def _():
        out_ref[0, 0] = 0.0

    @pl.when(flags_ref[i, j] != 0)
    def _():
        dn = (((0,), (0,)), ((), ()))
        hi = lax.Precision.HIGHEST
        d2 = lax.dot_general(cit_ref[...], cj_ref[...], dn, precision=hi)
        ri = lax.broadcasted_iota(jnp.int32, (_I_BLK, _J_BLK), 0) + i * _I_BLK
        rj = lax.broadcasted_iota(jnp.int32, (_I_BLK, _J_BLK), 1) + j * _J_BLK
        mask = (d2 < 1.0) & (ri != rj)
        dds = cj_ref[5:6, :]
        wcat = jnp.concatenate([w_ref[0], w_ref[1]], axis=1)
        hasm = jnp.full((_I_BLK, _J_BLK), -1.0, jnp.float32)
        for s in range(_MXU_S):
            r = lax.dot_general(pt_ref[s], wcat, dn, precision=hi)
            nv = r[:, :_J_BLK]
            nw = r[:, _J_BLK:]
            m = jnp.minimum(jnp.minimum(nv, nw), dds - (nv + nw))
            hasm = jnp.maximum(hasm, m)
        ax = w_ref[0, 0:1, :]
        ay = w_ref[0, 1:2, :]
        az = w_ref[0, 2:3, :]
        a4 = w_ref[0, 3:4, :]
        bx = w_ref[1, 0:1, :]
        by = w_ref[1, 1:2, :]
        bz = w_ref[1, 2:3, :]
        b4 = w_ref[1, 3:4, :]
        for s in range(_MXU_S, _NS):
            px = pc_ref[:, (s * 3 + 0):(s * 3 + 1)]
            py = pc_ref[:, (s * 3 + 1):(s * 3 + 2)]
            pz = pc_ref[:, (s * 3 + 2):(s * 3 + 3)]
            nv = px * ax + py * ay + pz * az + a4
            nw = px * bx + py * by + pz * bz + b4
            m = jnp.minimum(jnp.minimum(nv, nw), dds - (nv + nw))
            hasm = jnp.maximum(hasm, m)
        partial = jnp.sum(jnp.where(mask & (hasm >= 0.0), fp_ref[...], 0.0))
        out_ref[0, 0] = out_ref[0, 0] + partial

    @pl.when((i == _NI - 1) & (j == _NJ - 1))
    def _():
        out_ref[0, 0] = out_ref[0, 0] * (1.0 / 5000.0)


def _stage2(flags, PT, PC, W, CIT, CJ, fp2):
    out = pl.pallas_call(
        _tc_body,
        grid=(_NI, _NJ),
        in_specs=[
            pl.BlockSpec(memory_space=pltpu.SMEM),
            pl.BlockSpec((_NS, 4, _I_BLK), lambda i, j: (0, 0, i)),
            pl.BlockSpec((_I_BLK, _PCW), lambda i, j: (i, 0)),
            pl.BlockSpec((2, 4, _J_BLK), lambda i, j: (0, 0, j)),
            pl.BlockSpec((6, _I_BLK), lambda i, j: (0, i)),
            pl.BlockSpec((6, _J_BLK), lambda i, j: (0, j)),
            pl.BlockSpec((_I_BLK, 1), lambda i, j: (i, 0)),
        ],
        out_specs=pl.BlockSpec(memory_space=pltpu.SMEM),
        out_shape=jax.ShapeDtypeStruct((1, 1), jnp.float32),
    )(flags, PT, PC, W, CIT, CJ, fp2)
    return out[0, 0]


@jax.jit
def kernel(vertices, faces, face_probs):
    F = faces.shape[0]
    kk = jax.random.key(42)
    ka, kb = jax.random.split(kk)
    alpha = jax.random.uniform(ka, (F, _NS), dtype=jnp.float32)
    beta = jax.random.uniform(kb, (F, _NS), dtype=jnp.float32) * (1.0 - alpha)
    gamma = 1.0 - alpha - beta
    pad = _FPAD - F
    abg = jnp.pad(jnp.concatenate([alpha, beta, gamma], axis=1),
                  ((0, pad), (0, 128 - 3 * _NS)))
    facesP = jnp.pad(faces, ((0, pad), (0, 0)))
    f0 = facesP[:, 0]
    f1 = facesP[:, 1]
    f2 = facesP[:, 2]
    vx = vertices[:, 0]
    vy = vertices[:, 1]
    vz = vertices[:, 2]
    fpP = jnp.pad(face_probs, (0, pad))

    (keys,) = _sc_keys()(vx, f0, f1, f2)
    xs, perm = lax.sort((keys, lax.iota(jnp.int32, _FPAD)),
                        dimension=0, num_keys=1)
    xlo_i = xs[::_I_BLK]
    xhi_i = xs[_I_BLK - 1::_I_BLK]
    xlo_j = xs[::_J_BLK]
    xhi_j = xs[_J_BLK - 1::_J_BLK]
    eps = jnp.float32(1e-3)
    flags = ((xlo_j[None, :] <= xhi_i[:, None] + 1.0 + eps)
             & (xlo_i[:, None] <= xhi_j[None, :] + 1.0 + eps)).astype(jnp.int32)

    ptf, wf, citf, cjf, pt2f, fps = _sc_stage1()(
        vx, vy, vz, f0, f1, f2, fpP, abg, perm)
    PT = ptf.reshape(_NS, 4, _FPAD)
    PC = pt2f.reshape(_FPAD, _PCW)
    W = wf.reshape(2, 4, _FPAD)
    CIT = citf.reshape(6, _FPAD)
    CJ = cjf.reshape(6, _FPAD)
    fp2 = fps.reshape(_FPAD, 1)
    loss = _stage2(flags, PT, PC, W, CIT, CJ, fp2)
    return loss


# speedup vs baseline: 58.8741x; 1.3595x over previous
import functools
import jax
import jax.numpy as jnp
from jax import lax
from jax.experimental import pallas as pl
from jax.experimental.pallas import tpu as pltpu
from jax.experimental.pallas import tpu_sc as plsc

_NS = 10
_FPAD = 5120
_NV = 2500
_NC = 2
_NSUB = 16
_NW = _NC * _NSUB
_FPW = _FPAD // _NW
_CHUNKS = _FPW // 16
_PCW = 32

_I_BLK = 256
_J_BLK = 512
_NI = _FPAD // _I_BLK
_NJ = _FPAD // _J_BLK

_MXU_S = 1

_SC_MESH = plsc.VectorSubcoreMesh(
    core_axis_name="c", subcore_axis_name="s",
    num_cores=_NC, num_subcores=_NSUB)
_SC_PARAMS = pltpu.CompilerParams(needs_layout_passes=False)


def _keys_body(vx_h, f0_h, f1_h, f2_h, keys_h, vx, f0b, f1b, f2b, keysb):
    wid = lax.axis_index("s") * _NC + lax.axis_index("c")
    tb = wid * _FPW
    pltpu.sync_copy(vx_h, vx)
    pltpu.sync_copy(f0_h.at[pl.ds(tb, _FPW)], f0b)
    pltpu.sync_copy(f1_h.at[pl.ds(tb, _FPW)], f1b)
    pltpu.sync_copy(f2_h.at[pl.ds(tb, _FPW)], f2b)
    third = jnp.float32(1.0 / 3.0)

    def chunk(k, _):
        o = k * 16
        v1x = plsc.load_gather(vx, [f0b[pl.ds(o, 16)]])
        v2x = plsc.load_gather(vx, [f1b[pl.ds(o, 16)]])
        v3x = plsc.load_gather(vx, [f2b[pl.ds(o, 16)]])
        cx = (v1x + v2x + v3x) * third
        gid = tb + o + lax.iota(jnp.int32, 16)
        keysb[pl.ds(o, 16)] = jnp.where(gid < jnp.int32(5000), cx,
                                        jnp.float32(1e9))
        return ()

    lax.fori_loop(0, _CHUNKS, chunk, ())
    pltpu.sync_copy(keysb, keys_h.at[pl.ds(tb, _FPW)])


_sc_keys = functools.partial(
    pl.kernel,
    _keys_body,
    out_type=[jax.ShapeDtypeStruct((_FPAD,), jnp.float32)],
    mesh=_SC_MESH,
    compiler_params=_SC_PARAMS,
    scratch_types=[
        pltpu.VMEM((_NV,), jnp.float32),
        pltpu.VMEM((_FPW,), jnp.int32),
        pltpu.VMEM((_FPW,), jnp.int32),
        pltpu.VMEM((_FPW,), jnp.int32),
        pltpu.VMEM((_FPW,), jnp.float32),
    ],
)


def _sc_body(vx_h, vy_h, vz_h, f0_h, f1_h, f2_h, fp_h, abg_h, perm_h,
             pt_h, w_h, cit_h, cj_h, pt2_h, fps_h,
             vx, vy, vz, f0f, f1f, f2f, fpf, permb, abgb,
             ptb, wb, citb, cjb, ptb2, fpsb, sem):
    wid = lax.axis_index("s") * _NC + lax.axis_index("c")
    tb = wid * _FPW

    pltpu.sync_copy(vx_h, vx)
    pltpu.sync_copy(vy_h, vy)
    pltpu.sync_copy(vz_h, vz)
    pltpu.sync_copy(f0_h, f0f)
    pltpu.sync_copy(f1_h, f1f)
    pltpu.sync_copy(f2_h, f2f)
    pltpu.sync_copy(fp_h, fpf)
    pltpu.sync_copy(perm_h.at[pl.ds(tb, _FPW)], permb)
    copies = []
    for k in range(_CHUNKS):
        idxv = permb[pl.ds(k * 16, 16)]
        copies.append(pltpu.async_copy(
            abg_h.at[idxv], abgb.at[pl.ds(k * 16, 16)], sem))
    for c in copies:
        c.wait()

    one = jnp.full((16,), 1.0, jnp.float32)
    third = jnp.float32(1.0 / 3.0)

    def chunk(k, _):
        o = k * 16
        rows = o + lax.iota(jnp.int32, 16)
        pch = permb[pl.ds(o, 16)]
        f0v = plsc.load_gather(f0f, [pch])
        f1v = plsc.load_gather(f1f, [pch])
        f2v = plsc.load_gather(f2f, [pch])
        fpv = plsc.load_gather(fpf, [pch])
        v1x = plsc.load_gather(vx, [f0v])
        v1y = plsc.load_gather(vy, [f0v])
        v1z = plsc.load_gather(vz, [f0v])
        v2x = plsc.load_gather(vx, [f1v])
        v2y = plsc.load_gather(vy, [f1v])
        v2z = plsc.load_gather(vz, [f1v])
        v3x = plsc.load_gather(vx, [f2v])
        v3y = plsc.load_gather(vy, [f2v])
        v3z = plsc.load_gather(vz, [f2v])

        e0x = v2x - v1x
        e0y = v2y - v1y
        e0z = v2z - v1z
        e1x = v3x - v1x
        e1y = v3y - v1y
        e1z = v3z - v1z
        d00 = e0x * e0x + e0y * e0y + e0z * e0z
        d01 = e0x * e1x + e0y * e1y + e0z * e1z
        d11 = e1x * e1x + e1y * e1y + e1z * e1z
        den = d00 * d11 - d01 * d01
        c0 = v1x * e0x + v1y * e0y + v1z * e0z
        c1 = v1x * e1x + v1y * e1y + v1z * e1z
        sgn = jnp.where(den > 0.0, 1.0, -1.0).astype(jnp.float32)
        dnz = den != 0.0
        sd00 = sgn * d00
        sd01 = sgn * d01
        sd11 = sgn * d11
        ax = sd11 * e0x - sd01 * e1x
        ay = sd11 * e0y - sd01 * e1y
        az = sd11 * e0z - sd01 * e1z
        a4 = sd01 * c1 - sd11 * c0
        bx = sd00 * e1x - sd01 * e0x
        by = sd00 * e1y - sd01 * e0y
        bz = sd00 * e1z - sd01 * e0z
        b4 = sd01 * c0 - sd00 * c1
        dds = jnp.where(dnz, sgn * den, jnp.float32(-1.0))

        cenx = (v1x + v2x + v3x) * third
        ceny = (v1y + v2y + v3y) * third
        cenz = (v1z + v2z + v3z) * third
        sq = cenx * cenx + ceny * ceny + cenz * cenz
        sqm = jnp.where(pch < jnp.int32(5000), sq, jnp.float32(1e9))

        wb[pl.ds(0 * _FPW + o, 16)] = ax
        wb[pl.ds(1 * _FPW + o, 16)] = ay
        wb[pl.ds(2 * _FPW + o, 16)] = az
        wb[pl.ds(3 * _FPW + o, 16)] = a4
        wb[pl.ds(4 * _FPW + o, 16)] = bx
        wb[pl.ds(5 * _FPW + o, 16)] = by
        wb[pl.ds(6 * _FPW + o, 16)] = bz
        wb[pl.ds(7 * _FPW + o, 16)] = b4

        citb[pl.ds(0 * _FPW + o, 16)] = cenx
        citb[pl.ds(1 * _FPW + o, 16)] = ceny
        citb[pl.ds(2 * _FPW + o, 16)] = cenz
        citb[pl.ds(3 * _FPW + o, 16)] = sqm
        citb[pl.ds(4 * _FPW + o, 16)] = one
        citb[pl.ds(5 * _FPW + o, 16)] = jnp.zeros((16,), jnp.float32)

        m2 = jnp.float32(-2.0)
        cjb[pl.ds(0 * _FPW + o, 16)] = m2 * cenx
        cjb[pl.ds(1 * _FPW + o, 16)] = m2 * ceny
        cjb[pl.ds(2 * _FPW + o, 16)] = m2 * cenz
        cjb[pl.ds(3 * _FPW + o, 16)] = one
        cjb[pl.ds(4 * _FPW + o, 16)] = sqm
        cjb[pl.ds(5 * _FPW + o, 16)] = dds

        fpsb[pl.ds(o, 16)] = fpv

        rowbase = lax.iota(jnp.int32, 16) * _PCW + o * _PCW
        for s in range(_NS):
            colal = jnp.full((16,), s, jnp.int32)
            colbe = jnp.full((16,), _NS + s, jnp.int32)
            colga = jnp.full((16,), 2 * _NS + s, jnp.int32)
            al = plsc.load_gather(abgb, [rows, colal])
            be = plsc.load_gather(abgb, [rows, colbe])
            ga = plsc.load_gather(abgb, [rows, colga])
            px = al * v1x + be * v2x + ga * v3x
            py = al * v1y + be * v2y + ga * v3y
            pz = al * v1z + be * v2z + ga * v3z
            ptb[pl.ds((s * 4 + 0) * _FPW + o, 16)] = px
            ptb[pl.ds((s * 4 + 1) * _FPW + o, 16)] = py
            ptb[pl.ds((s * 4 + 2) * _FPW + o, 16)] = pz
            ptb[pl.ds((s * 4 + 3) * _FPW + o, 16)] = one
            plsc.store_scatter(ptb2, [rowbase + (s * 3 + 0)], px)
            plsc.store_scatter(ptb2, [rowbase + (s * 3 + 1)], py)
            plsc.store_scatter(ptb2, [rowbase + (s * 3 + 2)], pz)
        return ()

    lax.fori_loop(0, _CHUNKS, chunk, ())

    for r in range(_NS * 4):
        pltpu.sync_copy(ptb.at[pl.ds(r * _FPW, _FPW)],
                        pt_h.at[pl.ds(r * _FPAD + tb, _FPW)])
    for r in range(8):
        pltpu.sync_copy(wb.at[pl.ds(r * _FPW, _FPW)],
                        w_h.at[pl.ds(r * _FPAD + tb, _FPW)])
    for r in range(6):
        pltpu.sync_copy(citb.at[pl.ds(r * _FPW, _FPW)],
                        cit_h.at[pl.ds(r * _FPAD + tb, _FPW)])
        pltpu.sync_copy(cjb.at[pl.ds(r * _FPW, _FPW)],
                        cj_h.at[pl.ds(r * _FPAD + tb, _FPW)])
    pltpu.sync_copy(ptb2, pt2_h.at[pl.ds(tb * _PCW, _FPW * _PCW)])
    pltpu.sync_copy(fpsb, fps_h.at[pl.ds(tb, _FPW)])


_sc_stage1 = functools.partial(
    pl.kernel,
    _sc_body,
    out_type=[
        jax.ShapeDtypeStruct((_NS * 4 * _FPAD,), jnp.float32),
        jax.ShapeDtypeStruct((8 * _FPAD,), jnp.float32),
        jax.ShapeDtypeStruct((6 * _FPAD,), jnp.float32),
        jax.ShapeDtypeStruct((6 * _FPAD,), jnp.float32),
        jax.ShapeDtypeStruct((_FPAD * _PCW,), jnp.float32),
        jax.ShapeDtypeStruct((_FPAD,), jnp.float32),
    ],
    mesh=_SC_MESH,
    compiler_params=_SC_PARAMS,
    scratch_types=[
        pltpu.VMEM((_NV,), jnp.float32),
        pltpu.VMEM((_NV,), jnp.float32),
        pltpu.VMEM((_NV,), jnp.float32),
        pltpu.VMEM((_FPAD,), jnp.int32),
        pltpu.VMEM((_FPAD,), jnp.int32),
        pltpu.VMEM((_FPAD,), jnp.int32),
        pltpu.VMEM((_FPAD,), jnp.float32),
        pltpu.VMEM((_FPW,), jnp.int32),
        pltpu.VMEM((_FPW, 128), jnp.float32),
        pltpu.VMEM((_NS * 4 * _FPW,), jnp.float32),
        pltpu.VMEM((8 * _FPW,), jnp.float32),
        pltpu.VMEM((6 * _FPW,), jnp.float32),
        pltpu.VMEM((6 * _FPW,), jnp.float32),
        pltpu.VMEM((_FPW * _PCW,), jnp.float32),
        pltpu.VMEM((_FPW,), jnp.float32),
        pltpu.SemaphoreType.DMA,
    ],
)


def _tc_body(flags_ref, pt_ref, pc_ref, w_ref, cit_ref, cj_ref, fp_ref,
             out_ref):
    i = pl.program_id(0)
    j = pl.program_id(1)

    @pl.when((i == 0) & (j == 0))
    def _():
        out_ref[0, 0] = 0.0

    @pl.when(flags_ref[i, j] != 0)
    def _():
        dn = (((0,), (0,)), ((), ()))
        hi = lax.Precision.HIGHEST
        d2 = lax.dot_general(cit_ref[...], cj_ref[...], dn, precision=hi)
        ri = lax.broadcasted_iota(jnp.int32, (_I_BLK, _J_BLK), 0) + i * _I_BLK
        rj = lax.broadcasted_iota(jnp.int32, (_I_BLK, _J_BLK), 1) + j * _J_BLK
        mask = (d2 < 1.0) & (ri != rj)
        dds = cj_ref[5:6, :]
        wcat = jnp.concatenate([w_ref[0], w_ref[1]], axis=1)
        hasm = jnp.full((_I_BLK, _J_BLK), -1.0, jnp.float32)
        for s in range(_MXU_S):
            r = lax.dot_general(pt_ref[s], wcat, dn, precision=hi)
            nv = r[:, :_J_BLK]
            nw = r[:, _J_BLK:]
            m = jnp.minimum(jnp.minimum(nv, nw), dds - (nv + nw))
            hasm = jnp.maximum(hasm, m)
        ax = w_ref[0, 0:1, :]
        ay = w_ref[0, 1:2, :]
        az = w_ref[0, 2:3, :]
        a4 = w_ref[0, 3:4, :]
        bx = w_ref[1, 0:1, :]
        by = w_ref[1, 1:2, :]
        bz = w_ref[1, 2:3, :]
        b4 = w_ref[1, 3:4, :]
        for s in range(_MXU_S, _NS):
            px = pc_ref[:, (s * 3 + 0):(s * 3 + 1)]
            py = pc_ref[:, (s * 3 + 1):(s * 3 + 2)]
            pz = pc_ref[:, (s * 3 + 2):(s * 3 + 3)]
            nv = px * ax + py * ay + pz * az + a4
            nw = px * bx + py * by + pz * bz + b4
            m = jnp.minimum(jnp.minimum(nv, nw), dds - (nv + nw))
            hasm = jnp.maximum(hasm, m)
        partial = jnp.sum(jnp.where(mask & (hasm >= 0.0), fp_ref[...], 0.0))
        out_ref[0, 0] = out_ref[0, 0] + partial

    @pl.when((i == _NI - 1) & (j == _NJ - 1))
    def _():
        out_ref[0, 0] = out_ref[0, 0] * (1.0 / 5000.0)


def _stage2(flags, PT, PC, W, CIT, CJ, fp2):
    out = pl.pallas_call(
        _tc_body,
        grid=(_NI, _NJ),
        in_specs=[
            pl.BlockSpec(memory_space=pltpu.SMEM),
            pl.BlockSpec((_NS, 4, _I_BLK), lambda i, j: (0, 0, i)),
            pl.BlockSpec((_I_BLK, _PCW), lambda i, j: (i, 0)),
            pl.BlockSpec((2, 4, _J_BLK), lambda i, j: (0, 0, j)),
            pl.BlockSpec((6, _I_BLK), lambda i, j: (0, i)),
            pl.BlockSpec((6, _J_BLK), lambda i, j: (0, j)),
            pl.BlockSpec((_I_BLK, 1), lambda i, j: (i, 0)),
        ],
        out_specs=pl.BlockSpec(memory_space=pltpu.SMEM),
        out_shape=jax.ShapeDtypeStruct((1, 1), jnp.float32),
    )(flags, PT, PC, W, CIT, CJ, fp2)
    return out[0, 0]


@jax.jit
def kernel(vertices, faces, face_probs):
    F = faces.shape[0]
    kk = jax.random.key(42)
    ka, kb = jax.random.split(kk)
    alpha = jax.random.uniform(ka, (F, _NS), dtype=jnp.float32)
    beta = jax.random.uniform(kb, (F, _NS), dtype=jnp.float32) * (1.0 - alpha)
    gamma = 1.0 - alpha - beta
    pad = _FPAD - F
    abg = jnp.pad(jnp.concatenate([alpha, beta, gamma], axis=1),
                  ((0, pad), (0, 128 - 3 * _NS)))
    facesP = jnp.pad(faces, ((0, pad), (0, 0)))
    f0 = facesP[:, 0]
    f1 = facesP[:, 1]
    f2 = facesP[:, 2]
    vx = vertices[:, 0]
    vy = vertices[:, 1]
    vz = vertices[:, 2]
    fpP = jnp.pad(face_probs, (0, pad))

    (keys,) = _sc_keys()(vx, f0, f1, f2)
    xs, perm = lax.sort((keys, lax.iota(jnp.int32, _FPAD)),
                        dimension=0, num_keys=1)
    xlo_i = xs[::_I_BLK]
    xhi_i = xs[_I_BLK - 1::_I_BLK]
    xlo_j = xs[::_J_BLK]
    xhi_j = xs[_J_BLK - 1::_J_BLK]
    eps = jnp.float32(1e-3)
    flags = ((xlo_j[None, :] <= xhi_i[:, None] + 1.0 + eps)
             & (xlo_i[:, None] <= xhi_j[None, :] + 1.0 + eps)).astype(jnp.int32)

    ptf, wf, citf, cjf, pt2f, fps = _sc_stage1()(
        vx, vy, vz, f0, f1, f2, fpP, abg, perm)
    PT = ptf.reshape(_NS, 4, _FPAD)
    PC = pt2f.reshape(_FPAD, _PCW)
    W = wf.reshape(2, 4, _FPAD)
    CIT = citf.reshape(6, _FPAD)
    CJ = cjf.reshape(6, _FPAD)
    fp2 = fps.reshape(_FPAD, 1)
    loss = _stage2(flags, PT, PC, W, CIT, CJ, fp2)
    return loss


# speedup vs baseline: 59.7576x; 1.0150x over previous
import functools
import jax
import jax.numpy as jnp
from jax import lax
from jax.experimental import pallas as pl
from jax.experimental.pallas import tpu as pltpu
from jax.experimental.pallas import tpu_sc as plsc

_NS = 10
_FPAD = 5120
_NV = 2500
_NC = 2
_NSUB = 16
_NW = _NC * _NSUB
_FPW = _FPAD // _NW
_CHUNKS = _FPW // 16
_PCW = 32

_I_BLK = 256
_J_BLK = 512
_NI = _FPAD // _I_BLK
_NJ = _FPAD // _J_BLK

_MXU_S = 1

_SC_MESH = plsc.VectorSubcoreMesh(
    core_axis_name="c", subcore_axis_name="s",
    num_cores=_NC, num_subcores=_NSUB)
_SC_PARAMS = pltpu.CompilerParams(needs_layout_passes=False)


def _keys_body(vx_h, f0_h, f1_h, f2_h, keys_h, vx, f0b, f1b, f2b, keysb):
    wid = lax.axis_index("s") * _NC + lax.axis_index("c")
    tb = wid * _FPW
    pltpu.sync_copy(vx_h, vx)
    pltpu.sync_copy(f0_h.at[pl.ds(tb, _FPW)], f0b)
    pltpu.sync_copy(f1_h.at[pl.ds(tb, _FPW)], f1b)
    pltpu.sync_copy(f2_h.at[pl.ds(tb, _FPW)], f2b)
    third = jnp.float32(1.0 / 3.0)

    def chunk(k, _):
        o = k * 16
        v1x = plsc.load_gather(vx, [f0b[pl.ds(o, 16)]])
        v2x = plsc.load_gather(vx, [f1b[pl.ds(o, 16)]])
        v3x = plsc.load_gather(vx, [f2b[pl.ds(o, 16)]])
        cx = (v1x + v2x + v3x) * third
        gid = tb + o + lax.iota(jnp.int32, 16)
        keysb[pl.ds(o, 16)] = jnp.where(gid < jnp.int32(5000), cx,
                                        jnp.float32(1e9))
        return ()

    lax.fori_loop(0, _CHUNKS, chunk, ())
    pltpu.sync_copy(keysb, keys_h.at[pl.ds(tb, _FPW)])


_sc_keys = functools.partial(
    pl.kernel,
    _keys_body,
    out_type=[jax.ShapeDtypeStruct((_FPAD,), jnp.float32)],
    mesh=_SC_MESH,
    compiler_params=_SC_PARAMS,
    scratch_types=[
        pltpu.VMEM((_NV,), jnp.float32),
        pltpu.VMEM((_FPW,), jnp.int32),
        pltpu.VMEM((_FPW,), jnp.int32),
        pltpu.VMEM((_FPW,), jnp.int32),
        pltpu.VMEM((_FPW,), jnp.float32),
    ],
)


def _sc_body(vx_h, vy_h, vz_h, abg_h, perm_h,
             pt_h, w_h, cit_h, cj_h, pt2_h, fps_h,
             vx, vy, vz, permb, abgb,
             ptb, wb, citb, cjb, ptb2, fpsb, sem):
    wid = lax.axis_index("s") * _NC + lax.axis_index("c")
    tb = wid * _FPW

    pltpu.sync_copy(vx_h, vx)
    pltpu.sync_copy(vy_h, vy)
    pltpu.sync_copy(vz_h, vz)
    pltpu.sync_copy(perm_h.at[pl.ds(tb, _FPW)], permb)
    copies = []
    for k in range(_CHUNKS):
        idxv = permb[pl.ds(k * 16, 16)]
        copies.append(pltpu.async_copy(
            abg_h.at[idxv], abgb.at[pl.ds(k * 16, 16)], sem))
    for c in copies:
        c.wait()

    one = jnp.full((16,), 1.0, jnp.float32)
    third = jnp.float32(1.0 / 3.0)

    def chunk(k, _):
        o = k * 16
        rows = o + lax.iota(jnp.int32, 16)
        pch = permb[pl.ds(o, 16)]
        c30 = jnp.full((16,), 30, jnp.int32)
        f0v = plsc.bitcast(plsc.load_gather(abgb, [rows, c30]), jnp.int32)
        f1v = plsc.bitcast(plsc.load_gather(abgb, [rows, c30 + 1]), jnp.int32)
        f2v = plsc.bitcast(plsc.load_gather(abgb, [rows, c30 + 2]), jnp.int32)
        fpv = plsc.load_gather(abgb, [rows, c30 + 3])
        v1x = plsc.load_gather(vx, [f0v])
        v1y = plsc.load_gather(vy, [f0v])
        v1z = plsc.load_gather(vz, [f0v])
        v2x = plsc.load_gather(vx, [f1v])
        v2y = plsc.load_gather(vy, [f1v])
        v2z = plsc.load_gather(vz, [f1v])
        v3x = plsc.load_gather(vx, [f2v])
        v3y = plsc.load_gather(vy, [f2v])
        v3z = plsc.load_gather(vz, [f2v])

        e0x = v2x - v1x
        e0y = v2y - v1y
        e0z = v2z - v1z
        e1x = v3x - v1x
        e1y = v3y - v1y
        e1z = v3z - v1z
        d00 = e0x * e0x + e0y * e0y + e0z * e0z
        d01 = e0x * e1x + e0y * e1y + e0z * e1z
        d11 = e1x * e1x + e1y * e1y + e1z * e1z
        den = d00 * d11 - d01 * d01
        c0 = v1x * e0x + v1y * e0y + v1z * e0z
        c1 = v1x * e1x + v1y * e1y + v1z * e1z
        sgn = jnp.where(den > 0.0, 1.0, -1.0).astype(jnp.float32)
        dnz = den != 0.0
        sd00 = sgn * d00
        sd01 = sgn * d01
        sd11 = sgn * d11
        ax = sd11 * e0x - sd01 * e1x
        ay = sd11 * e0y - sd01 * e1y
        az = sd11 * e0z - sd01 * e1z
        a4 = sd01 * c1 - sd11 * c0
        bx = sd00 * e1x - sd01 * e0x
        by = sd00 * e1y - sd01 * e0y
        bz = sd00 * e1z - sd01 * e0z
        b4 = sd01 * c0 - sd00 * c1
        dds = jnp.where(dnz, sgn * den, jnp.float32(-1.0))

        cenx = (v1x + v2x + v3x) * third
        ceny = (v1y + v2y + v3y) * third
        cenz = (v1z + v2z + v3z) * third
        sq = cenx * cenx + ceny * ceny + cenz * cenz
        sqm = jnp.where(pch < jnp.int32(5000), sq, jnp.float32(1e9))

        wb[pl.ds(0 * _FPW + o, 16)] = ax
        wb[pl.ds(1 * _FPW + o, 16)] = ay
        wb[pl.ds(2 * _FPW + o, 16)] = az
        wb[pl.ds(3 * _FPW + o, 16)] = a4
        wb[pl.ds(4 * _FPW + o, 16)] = bx
        wb[pl.ds(5 * _FPW + o, 16)] = by
        wb[pl.ds(6 * _FPW + o, 16)] = bz
        wb[pl.ds(7 * _FPW + o, 16)] = b4

        citb[pl.ds(0 * _FPW + o, 16)] = cenx
        citb[pl.ds(1 * _FPW + o, 16)] = ceny
        citb[pl.ds(2 * _FPW + o, 16)] = cenz
        citb[pl.ds(3 * _FPW + o, 16)] = sqm
        citb[pl.ds(4 * _FPW + o, 16)] = one
        citb[pl.ds(5 * _FPW + o, 16)] = jnp.zeros((16,), jnp.float32)

        m2 = jnp.float32(-2.0)
        cjb[pl.ds(0 * _FPW + o, 16)] = m2 * cenx
        cjb[pl.ds(1 * _FPW + o, 16)] = m2 * ceny
        cjb[pl.ds(2 * _FPW + o, 16)] = m2 * cenz
        cjb[pl.ds(3 * _FPW + o, 16)] = one
        cjb[pl.ds(4 * _FPW + o, 16)] = sqm
        cjb[pl.ds(5 * _FPW + o, 16)] = dds

        fpsb[pl.ds(o, 16)] = fpv

        rowbase = lax.iota(jnp.int32, 16) * _PCW + o * _PCW
        for s in range(_NS):
            colal = jnp.full((16,), s, jnp.int32)
            colbe = jnp.full((16,), _NS + s, jnp.int32)
            colga = jnp.full((16,), 2 * _NS + s, jnp.int32)
            al = plsc.load_gather(abgb, [rows, colal])
            be = plsc.load_gather(abgb, [rows, colbe])
            ga = plsc.load_gather(abgb, [rows, colga])
            px = al * v1x + be * v2x + ga * v3x
            py = al * v1y + be * v2y + ga * v3y
            pz = al * v1z + be * v2z + ga * v3z
            ptb[pl.ds((s * 4 + 0) * _FPW + o, 16)] = px
            ptb[pl.ds((s * 4 + 1) * _FPW + o, 16)] = py
            ptb[pl.ds((s * 4 + 2) * _FPW + o, 16)] = pz
            ptb[pl.ds((s * 4 + 3) * _FPW + o, 16)] = one
            plsc.store_scatter(ptb2, [rowbase + (s * 3 + 0)], px)
            plsc.store_scatter(ptb2, [rowbase + (s * 3 + 1)], py)
            plsc.store_scatter(ptb2, [rowbase + (s * 3 + 2)], pz)
        return ()

    lax.fori_loop(0, _CHUNKS, chunk, ())

    drains = []
    for r in range(_NS * 4):
        drains.append(pltpu.async_copy(
            ptb.at[pl.ds(r * _FPW, _FPW)], pt_h.at[pl.ds(r * _FPAD + tb, _FPW)], sem))
    for r in range(8):
        drains.append(pltpu.async_copy(
            wb.at[pl.ds(r * _FPW, _FPW)], w_h.at[pl.ds(r * _FPAD + tb, _FPW)], sem))
    for r in range(6):
        drains.append(pltpu.async_copy(
            citb.at[pl.ds(r * _FPW, _FPW)], cit_h.at[pl.ds(r * _FPAD + tb, _FPW)], sem))
        drains.append(pltpu.async_copy(
            cjb.at[pl.ds(r * _FPW, _FPW)], cj_h.at[pl.ds(r * _FPAD + tb, _FPW)], sem))
    drains.append(pltpu.async_copy(
        ptb2, pt2_h.at[pl.ds(tb * _PCW, _FPW * _PCW)], sem))
    drains.append(pltpu.async_copy(
        fpsb, fps_h.at[pl.ds(tb, _FPW)], sem))
    for c in drains:
        c.wait()


_sc_stage1 = functools.partial(
    pl.kernel,
    _sc_body,
    out_type=[
        jax.ShapeDtypeStruct((_NS * 4 * _FPAD,), jnp.float32),
        jax.ShapeDtypeStruct((8 * _FPAD,), jnp.float32),
        jax.ShapeDtypeStruct((6 * _FPAD,), jnp.float32),
        jax.ShapeDtypeStruct((6 * _FPAD,), jnp.float32),
        jax.ShapeDtypeStruct((_FPAD * _PCW,), jnp.float32),
        jax.ShapeDtypeStruct((_FPAD,), jnp.float32),
    ],
    mesh=_SC_MESH,
    compiler_params=_SC_PARAMS,
    scratch_types=[
        pltpu.VMEM((_NV,), jnp.float32),
        pltpu.VMEM((_NV,), jnp.float32),
        pltpu.VMEM((_NV,), jnp.float32),
        pltpu.VMEM((_FPW,), jnp.int32),
        pltpu.VMEM((_FPW, 128), jnp.float32),
        pltpu.VMEM((_NS * 4 * _FPW,), jnp.float32),
        pltpu.VMEM((8 * _FPW,), jnp.float32),
        pltpu.VMEM((6 * _FPW,), jnp.float32),
        pltpu.VMEM((6 * _FPW,), jnp.float32),
        pltpu.VMEM((_FPW * _PCW,), jnp.float32),
        pltpu.VMEM((_FPW,), jnp.float32),
        pltpu.SemaphoreType.DMA,
    ],
)


def _tc_body(flags_ref, pt_ref, pc_ref, w_ref, cit_ref, cj_ref, fp_ref,
             out_ref):
    i = pl.program_id(0)
    j = pl.program_id(1)

    @pl.when((i == 0) & (j == 0))
    def _():
        out_ref[0, 0] = 0.0

    @pl.when(flags_ref[i, j] != 0)
    def _():
        dn = (((0,), (0,)), ((), ()))
        hi = lax.Precision.HIGHEST
        d2 = lax.dot_general(cit_ref[...], cj_ref[...], dn, precision=hi)
        ri = lax.broadcasted_iota(jnp.int32, (_I_BLK, _J_BLK), 0) + i * _I_BLK
        rj = lax.broadcasted_iota(jnp.int32, (_I_BLK, _J_BLK), 1) + j * _J_BLK
        mask = (d2 < 1.0) & (ri != rj)
        dds = cj_ref[5:6, :]
        wcat = jnp.concatenate([w_ref[0], w_ref[1]], axis=1)
        hasm = jnp.full((_I_BLK, _J_BLK), -1.0, jnp.float32)
        for s in range(_MXU_S):
            r = lax.dot_general(pt_ref[s], wcat, dn, precision=hi)
            nv = r[:, :_J_BLK]
            nw = r[:, _J_BLK:]
            m = jnp.minimum(jnp.minimum(nv, nw), dds - (nv + nw))
            hasm = jnp.maximum(hasm, m)
        ax = w_ref[0, 0:1, :]
        ay = w_ref[0, 1:2, :]
        az = w_ref[0, 2:3, :]
        a4 = w_ref[0, 3:4, :]
        bx = w_ref[1, 0:1, :]
        by = w_ref[1, 1:2, :]
        bz = w_ref[1, 2:3, :]
        b4 = w_ref[1, 3:4, :]
        for s in range(_MXU_S, _NS):
            px = pc_ref[:, (s * 3 + 0):(s * 3 + 1)]
            py = pc_ref[:, (s * 3 + 1):(s * 3 + 2)]
            pz = pc_ref[:, (s * 3 + 2):(s * 3 + 3)]
            nv = px * ax + py * ay + pz * az + a4
            nw = px * bx + py * by + pz * bz + b4
            m = jnp.minimum(jnp.minimum(nv, nw), dds - (nv + nw))
            hasm = jnp.maximum(hasm, m)
        partial = jnp.sum(jnp.where(mask & (hasm >= 0.0), fp_ref[...], 0.0))
        out_ref[0, 0] = out_ref[0, 0] + partial

    @pl.when((i == _NI - 1) & (j == _NJ - 1))
    def _():
        out_ref[0, 0] = out_ref[0, 0] * (1.0 / 5000.0)


def _stage2(flags, PT, PC, W, CIT, CJ, fp2):
    out = pl.pallas_call(
        _tc_body,
        grid=(_NI, _NJ),
        in_specs=[
            pl.BlockSpec(memory_space=pltpu.SMEM),
            pl.BlockSpec((_NS, 4, _I_BLK), lambda i, j: (0, 0, i)),
            pl.BlockSpec((_I_BLK, _PCW), lambda i, j: (i, 0)),
            pl.BlockSpec((2, 4, _J_BLK), lambda i, j: (0, 0, j)),
            pl.BlockSpec((6, _I_BLK), lambda i, j: (0, i)),
            pl.BlockSpec((6, _J_BLK), lambda i, j: (0, j)),
            pl.BlockSpec((_I_BLK, 1), lambda i, j: (i, 0)),
        ],
        out_specs=pl.BlockSpec(memory_space=pltpu.SMEM),
        out_shape=jax.ShapeDtypeStruct((1, 1), jnp.float32),
    )(flags, PT, PC, W, CIT, CJ, fp2)
    return out[0, 0]


@jax.jit
def kernel(vertices, faces, face_probs):
    F = faces.shape[0]
    kk = jax.random.key(42)
    ka, kb = jax.random.split(kk)
    alpha = jax.random.uniform(ka, (F, _NS), dtype=jnp.float32)
    beta = jax.random.uniform(kb, (F, _NS), dtype=jnp.float32) * (1.0 - alpha)
    gamma = 1.0 - alpha - beta
    pad = _FPAD - F
    abg = jnp.pad(
        jnp.concatenate(
            [alpha, beta, gamma,
             lax.bitcast_convert_type(faces, jnp.float32),
             face_probs[:, None]], axis=1),
        ((0, pad), (0, 128 - 3 * _NS - 4)))
    facesP = jnp.pad(faces, ((0, pad), (0, 0)))
    f0 = facesP[:, 0]
    f1 = facesP[:, 1]
    f2 = facesP[:, 2]
    vx = vertices[:, 0]
    vy = vertices[:, 1]
    vz = vertices[:, 2]
    fpP = jnp.pad(face_probs, (0, pad))

    (keys,) = _sc_keys()(vx, f0, f1, f2)
    xs, perm = lax.sort((keys, lax.iota(jnp.int32, _FPAD)),
                        dimension=0, num_keys=1)
    xlo_i = xs[::_I_BLK]
    xhi_i = xs[_I_BLK - 1::_I_BLK]
    xlo_j = xs[::_J_BLK]
    xhi_j = xs[_J_BLK - 1::_J_BLK]
    eps = jnp.float32(1e-3)
    flags = ((xlo_j[None, :] <= xhi_i[:, None] + 1.0 + eps)
             & (xlo_i[:, None] <= xhi_j[None, :] + 1.0 + eps)).astype(jnp.int32)

    ptf, wf, citf, cjf, pt2f, fps = _sc_stage1()(vx, vy, vz, abg, perm)
    PT = ptf.reshape(_NS, 4, _FPAD)
    PC = pt2f.reshape(_FPAD, _PCW)
    W = wf.reshape(2, 4, _FPAD)
    CIT = citf.reshape(6, _FPAD)
    CJ = cjf.reshape(6, _FPAD)
    fp2 = fps.reshape(_FPAD, 1)
    loss = _stage2(flags, PT, PC, W, CIT, CJ, fp2)
    return loss


# speedup vs baseline: 60.1793x; 1.0071x over previous
import functools
import jax
import jax.numpy as jnp
from jax import lax
from jax.experimental import pallas as pl
from jax.experimental.pallas import tpu as pltpu
from jax.experimental.pallas import tpu_sc as plsc

_NS = 10
_FPAD = 5120
_NV = 2500
_NC = 2
_NSUB = 16
_NW = _NC * _NSUB
_FPW = _FPAD // _NW
_CHUNKS = _FPW // 16
_PCW = 32

_I_BLK = 256
_J_BLK = 512
_NI = _FPAD // _I_BLK
_NJ = _FPAD // _J_BLK

_SC_MESH = plsc.VectorSubcoreMesh(
    core_axis_name="c", subcore_axis_name="s",
    num_cores=_NC, num_subcores=_NSUB)
_SC_PARAMS = pltpu.CompilerParams(needs_layout_passes=False)


def _keys_body(vx_h, f0_h, f1_h, f2_h, keys_h, vx, f0b, f1b, f2b, keysb):
    wid = lax.axis_index("s") * _NC + lax.axis_index("c")
    tb = wid * _FPW
    pltpu.sync_copy(vx_h, vx)
    pltpu.sync_copy(f0_h.at[pl.ds(tb, _FPW)], f0b)
    pltpu.sync_copy(f1_h.at[pl.ds(tb, _FPW)], f1b)
    pltpu.sync_copy(f2_h.at[pl.ds(tb, _FPW)], f2b)
    third = jnp.float32(1.0 / 3.0)

    def chunk(k, _):
        o = k * 16
        v1x = plsc.load_gather(vx, [f0b[pl.ds(o, 16)]])
        v2x = plsc.load_gather(vx, [f1b[pl.ds(o, 16)]])
        v3x = plsc.load_gather(vx, [f2b[pl.ds(o, 16)]])
        cx = (v1x + v2x + v3x) * third
        gid = tb + o + lax.iota(jnp.int32, 16)
        keysb[pl.ds(o, 16)] = jnp.where(gid < jnp.int32(5000), cx,
                                        jnp.float32(1e9))
        return ()

    lax.fori_loop(0, _CHUNKS, chunk, ())
    pltpu.sync_copy(keysb, keys_h.at[pl.ds(tb, _FPW)])


_sc_keys = functools.partial(
    pl.kernel,
    _keys_body,
    out_type=[jax.ShapeDtypeStruct((_FPAD,), jnp.float32)],
    mesh=_SC_MESH,
    compiler_params=_SC_PARAMS,
    scratch_types=[
        pltpu.VMEM((_NV,), jnp.float32),
        pltpu.VMEM((_FPW,), jnp.int32),
        pltpu.VMEM((_FPW,), jnp.int32),
        pltpu.VMEM((_FPW,), jnp.int32),
        pltpu.VMEM((_FPW,), jnp.float32),
    ],
)


def _sc_body(vx_h, vy_h, vz_h, abg_h, perm_h,
             w_h, cit_h, cj_h, pt2_h, fps_h,
             vx, vy, vz, permb, abgb,
             wb, citb, cjb, ptb2, fpsb, sem):
    wid = lax.axis_index("s") * _NC + lax.axis_index("c")
    tb = wid * _FPW

    pltpu.sync_copy(vx_h, vx)
    pltpu.sync_copy(vy_h, vy)
    pltpu.sync_copy(vz_h, vz)
    pltpu.sync_copy(perm_h.at[pl.ds(tb, _FPW)], permb)
    copies = []
    for k in range(_CHUNKS):
        idxv = permb[pl.ds(k * 16, 16)]
        copies.append(pltpu.async_copy(
            abg_h.at[idxv], abgb.at[pl.ds(k * 16, 16)], sem))
    for c in copies:
        c.wait()

    one = jnp.full((16,), 1.0, jnp.float32)
    third = jnp.float32(1.0 / 3.0)

    def chunk(k, _):
        o = k * 16
        rows = o + lax.iota(jnp.int32, 16)
        pch = permb[pl.ds(o, 16)]
        c30 = jnp.full((16,), 30, jnp.int32)
        f0v = plsc.bitcast(plsc.load_gather(abgb, [rows, c30]), jnp.int32)
        f1v = plsc.bitcast(plsc.load_gather(abgb, [rows, c30 + 1]), jnp.int32)
        f2v = plsc.bitcast(plsc.load_gather(abgb, [rows, c30 + 2]), jnp.int32)
        fpv = plsc.load_gather(abgb, [rows, c30 + 3])
        v1x = plsc.load_gather(vx, [f0v])
        v1y = plsc.load_gather(vy, [f0v])
        v1z = plsc.load_gather(vz, [f0v])
        v2x = plsc.load_gather(vx, [f1v])
        v2y = plsc.load_gather(vy, [f1v])
        v2z = plsc.load_gather(vz, [f1v])
        v3x = plsc.load_gather(vx, [f2v])
        v3y = plsc.load_gather(vy, [f2v])
        v3z = plsc.load_gather(vz, [f2v])

        e0x = v2x - v1x
        e0y = v2y - v1y
        e0z = v2z - v1z
        e1x = v3x - v1x
        e1y = v3y - v1y
        e1z = v3z - v1z
        d00 = e0x * e0x + e0y * e0y + e0z * e0z
        d01 = e0x * e1x + e0y * e1y + e0z * e1z
        d11 = e1x * e1x + e1y * e1y + e1z * e1z
        den = d00 * d11 - d01 * d01
        c0 = v1x * e0x + v1y * e0y + v1z * e0z
        c1 = v1x * e1x + v1y * e1y + v1z * e1z
        sgn = jnp.where(den > 0.0, 1.0, -1.0).astype(jnp.float32)
        dnz = den != 0.0
        sd00 = sgn * d00
        sd01 = sgn * d01
        sd11 = sgn * d11
        ax = sd11 * e0x - sd01 * e1x
        ay = sd11 * e0y - sd01 * e1y
        az = sd11 * e0z - sd01 * e1z
        a4 = sd01 * c1 - sd11 * c0
        bx = sd00 * e1x - sd01 * e0x
        by = sd00 * e1y - sd01 * e0y
        bz = sd00 * e1z - sd01 * e0z
        b4 = sd01 * c0 - sd00 * c1
        dds = jnp.where(dnz, sgn * den, jnp.float32(-1.0))

        cenx = (v1x + v2x + v3x) * third
        ceny = (v1y + v2y + v3y) * third
        cenz = (v1z + v2z + v3z) * third
        sq = cenx * cenx + ceny * ceny + cenz * cenz
        sqm = jnp.where(pch < jnp.int32(5000), sq, jnp.float32(1e9))

        wb[pl.ds(0 * _FPW + o, 16)] = ax
        wb[pl.ds(1 * _FPW + o, 16)] = ay
        wb[pl.ds(2 * _FPW + o, 16)] = az
        wb[pl.ds(3 * _FPW + o, 16)] = a4
        wb[pl.ds(4 * _FPW + o, 16)] = bx
        wb[pl.ds(5 * _FPW + o, 16)] = by
        wb[pl.ds(6 * _FPW + o, 16)] = bz
        wb[pl.ds(7 * _FPW + o, 16)] = b4

        citb[pl.ds(0 * _FPW + o, 16)] = cenx
        citb[pl.ds(1 * _FPW + o, 16)] = ceny
        citb[pl.ds(2 * _FPW + o, 16)] = cenz
        citb[pl.ds(3 * _FPW + o, 16)] = sqm
        citb[pl.ds(4 * _FPW + o, 16)] = one
        citb[pl.ds(5 * _FPW + o, 16)] = jnp.zeros((16,), jnp.float32)

        m2 = jnp.float32(-2.0)
        cjb[pl.ds(0 * _FPW + o, 16)] = m2 * cenx
        cjb[pl.ds(1 * _FPW + o, 16)] = m2 * ceny
        cjb[pl.ds(2 * _FPW + o, 16)] = m2 * cenz
        cjb[pl.ds(3 * _FPW + o, 16)] = one
        cjb[pl.ds(4 * _FPW + o, 16)] = sqm
        cjb[pl.ds(5 * _FPW + o, 16)] = dds

        fpsb[pl.ds(o, 16)] = fpv

        rowbase = lax.iota(jnp.int32, 16) * _PCW + o * _PCW
        for s in range(_NS):
            colal = jnp.full((16,), s, jnp.int32)
            colbe = jnp.full((16,), _NS + s, jnp.int32)
            colga = jnp.full((16,), 2 * _NS + s, jnp.int32)
            al = plsc.load_gather(abgb, [rows, colal])
            be = plsc.load_gather(abgb, [rows, colbe])
            ga = plsc.load_gather(abgb, [rows, colga])
            px = al * v1x + be * v2x + ga * v3x
            py = al * v1y + be * v2y + ga * v3y
            pz = al * v1z + be * v2z + ga * v3z
            plsc.store_scatter(ptb2, [rowbase + (s * 3 + 0)], px)
            plsc.store_scatter(ptb2, [rowbase + (s * 3 + 1)], py)
            plsc.store_scatter(ptb2, [rowbase + (s * 3 + 2)], pz)
        return ()

    lax.fori_loop(0, _CHUNKS, chunk, ())

    drains = []
    for r in range(8):
        drains.append(pltpu.async_copy(
            wb.at[pl.ds(r * _FPW, _FPW)], w_h.at[pl.ds(r * _FPAD + tb, _FPW)], sem))
    for r in range(6):
        drains.append(pltpu.async_copy(
            citb.at[pl.ds(r * _FPW, _FPW)], cit_h.at[pl.ds(r * _FPAD + tb, _FPW)], sem))
        drains.append(pltpu.async_copy(
            cjb.at[pl.ds(r * _FPW, _FPW)], cj_h.at[pl.ds(r * _FPAD + tb, _FPW)], sem))
    drains.append(pltpu.async_copy(
        ptb2, pt2_h.at[pl.ds(tb * _PCW, _FPW * _PCW)], sem))
    drains.append(pltpu.async_copy(
        fpsb, fps_h.at[pl.ds(tb, _FPW)], sem))
    for c in drains:
        c.wait()


_sc_stage1 = functools.partial(
    pl.kernel,
    _sc_body,
    out_type=[
        jax.ShapeDtypeStruct((8 * _FPAD,), jnp.float32),
        jax.ShapeDtypeStruct((6 * _FPAD,), jnp.float32),
        jax.ShapeDtypeStruct((6 * _FPAD,), jnp.float32),
        jax.ShapeDtypeStruct((_FPAD * _PCW,), jnp.float32),
        jax.ShapeDtypeStruct((_FPAD,), jnp.float32),
    ],
    mesh=_SC_MESH,
    compiler_params=_SC_PARAMS,
    scratch_types=[
        pltpu.VMEM((_NV,), jnp.float32),
        pltpu.VMEM((_NV,), jnp.float32),
        pltpu.VMEM((_NV,), jnp.float32),
        pltpu.VMEM((_FPW,), jnp.int32),
        pltpu.VMEM((_FPW, 128), jnp.float32),
        pltpu.VMEM((8 * _FPW,), jnp.float32),
        pltpu.VMEM((6 * _FPW,), jnp.float32),
        pltpu.VMEM((6 * _FPW,), jnp.float32),
        pltpu.VMEM((_FPW * _PCW,), jnp.float32),
        pltpu.VMEM((_FPW,), jnp.float32),
        pltpu.SemaphoreType.DMA,
    ],
)


def _tc_body(flags_ref, pc_ref, w_ref, cit_ref, cj_ref, fp_ref,
             out_ref):
    i = pl.program_id(0)
    j = pl.program_id(1)

    @pl.when((i == 0) & (j == 0))
    def _():
        out_ref[0, 0] = 0.0

    @pl.when(flags_ref[i, j] != 0)
    def _():
        dn = (((0,), (0,)), ((), ()))
        hi = lax.Precision.HIGHEST
        d2 = lax.dot_general(cit_ref[...], cj_ref[...], dn, precision=hi)
        ri = lax.broadcasted_iota(jnp.int32, (_I_BLK, _J_BLK), 0) + i * _I_BLK
        rj = lax.broadcasted_iota(jnp.int32, (_I_BLK, _J_BLK), 1) + j * _J_BLK
        mask = (d2 < 1.0) & (ri != rj)
        dds = cj_ref[5:6, :]
        hasm = jnp.full((_I_BLK, _J_BLK), -1.0, jnp.float32)
        ax = w_ref[0, 0:1, :]
        ay = w_ref[0, 1:2, :]
        az = w_ref[0, 2:3, :]
        a4 = w_ref[0, 3:4, :]
        bx = w_ref[1, 0:1, :]
        by = w_ref[1, 1:2, :]
        bz = w_ref[1, 2:3, :]
        b4 = w_ref[1, 3:4, :]
        for s in range(_NS):
            px = pc_ref[:, (s * 3 + 0):(s * 3 + 1)]
            py = pc_ref[:, (s * 3 + 1):(s * 3 + 2)]
            pz = pc_ref[:, (s * 3 + 2):(s * 3 + 3)]
            nv = px * ax + py * ay + pz * az + a4
            nw = px * bx + py * by + pz * bz + b4
            m = jnp.minimum(jnp.minimum(nv, nw), dds - (nv + nw))
            hasm = jnp.maximum(hasm, m)
        partial = jnp.sum(jnp.where(mask & (hasm >= 0.0), fp_ref[...], 0.0))
        out_ref[0, 0] = out_ref[0, 0] + partial

    @pl.when((i == _NI - 1) & (j == _NJ - 1))
    def _():
        out_ref[0, 0] = out_ref[0, 0] * (1.0 / 5000.0)


def _stage2(flags, PC, W, CIT, CJ, fp2):
    out = pl.pallas_call(
        _tc_body,
        grid=(_NI, _NJ),
        in_specs=[
            pl.BlockSpec(memory_space=pltpu.SMEM),
            pl.BlockSpec((_I_BLK, _PCW), lambda i, j: (i, 0)),
            pl.BlockSpec((2, 4, _J_BLK), lambda i, j: (0, 0, j)),
            pl.BlockSpec((6, _I_BLK), lambda i, j: (0, i)),
            pl.BlockSpec((6, _J_BLK), lambda i, j: (0, j)),
            pl.BlockSpec((_I_BLK, 1), lambda i, j: (i, 0)),
        ],
        out_specs=pl.BlockSpec(memory_space=pltpu.SMEM),
        out_shape=jax.ShapeDtypeStruct((1, 1), jnp.float32),
    )(flags, PC, W, CIT, CJ, fp2)
    return out[0, 0]


@jax.jit
def kernel(vertices, faces, face_probs):
    F = faces.shape[0]
    kk = jax.random.key(42)
    ka, kb = jax.random.split(kk)
    alpha = jax.random.uniform(ka, (F, _NS), dtype=jnp.float32)
    beta = jax.random.uniform(kb, (F, _NS), dtype=jnp.float32) * (1.0 - alpha)
    gamma = 1.0 - alpha - beta
    pad = _FPAD - F
    abg = jnp.pad(
        jnp.concatenate(
            [alpha, beta, gamma,
             lax.bitcast_convert_type(faces, jnp.float32),
             face_probs[:, None]], axis=1),
        ((0, pad), (0, 128 - 3 * _NS - 4)))
    facesP = jnp.pad(faces, ((0, pad), (0, 0)))
    f0 = facesP[:, 0]
    f1 = facesP[:, 1]
    f2 = facesP[:, 2]
    vx = vertices[:, 0]
    vy = vertices[:, 1]
    vz = vertices[:, 2]
    fpP = jnp.pad(face_probs, (0, pad))

    (keys,) = _sc_keys()(vx, f0, f1, f2)
    xs, perm = lax.sort((keys, lax.iota(jnp.int32, _FPAD)),
                        dimension=0, num_keys=1)
    xlo_i = xs[::_I_BLK]
    xhi_i = xs[_I_BLK - 1::_I_BLK]
    xlo_j = xs[::_J_BLK]
    xhi_j = xs[_J_BLK - 1::_J_BLK]
    eps = jnp.float32(1e-3)
    flags = ((xlo_j[None, :] <= xhi_i[:, None] + 1.0 + eps)
             & (xlo_i[:, None] <= xhi_j[None, :] + 1.0 + eps)).astype(jnp.int32)

    wf, citf, cjf, pt2f, fps = _sc_stage1()(vx, vy, vz, abg, perm)
    PC = pt2f.reshape(_FPAD, _PCW)
    W = wf.reshape(2, 4, _FPAD)
    CIT = citf.reshape(6, _FPAD)
    CJ = cjf.reshape(6, _FPAD)
    fp2 = fps.reshape(_FPAD, 1)
    loss = _stage2(flags, PC, W, CIT, CJ, fp2)
    return loss
